# baseline (speedup 1.0000x reference)
import functools

import jax
import jax.numpy as jnp
from jax import lax
from jax.experimental import pallas as pl
from jax.experimental.pallas import tpu as pltpu
from jax.experimental.pallas import tpu_sc as plsc

NC = 2
NS = 16
NW = NC * NS

N_NODES = 10000
HALF = 5040
ACC_ROWS = HALF + 8
SUB_ROWS = 320
SUB_ROWS_LAST = HALF - (NS - 1) * SUB_ROWS
N_PAD = 10240
E_EDGES = 320000
CHUNK = 128
CHUNKS_PER_TILE = 160
E_PAD = NS * CHUNKS_PER_TILE * CHUNK
PAD_DST = 1 << 20
CNT_PAD = 5120
D = 128


def _remap_dst(dst_v, node_base):
    def remap(i, _):
        for j in range(CHUNK // 16):
            d = dst_v[i, pl.ds(j * 16, 16)] - node_base
            ok = (d >= 0) & (d < HALF)
            dst_v[i, pl.ds(j * 16, 16)] = jnp.where(ok, d, HALF)
        return 0
    lax.fori_loop(0, CHUNKS_PER_TILE, remap, 0)


def _acc_zero(acc_sh, zbuf, off, n):
    done = 0
    while n - done >= CHUNK:
        pltpu.sync_copy(zbuf, acc_sh.at[pl.ds(off + done, CHUNK)])
        done += CHUNK
    if n > done:
        pltpu.sync_copy(zbuf.at[pl.ds(0, n - done)],
                        acc_sh.at[pl.ds(off + done, n - done)])


def _acc_writeback(acc_sh, buf, out_hbm, acc_off, out_off, n):
    done = 0
    while done < n:
        m = min(CHUNK, n - done)
        pltpu.sync_copy(acc_sh.at[pl.ds(acc_off + done, m)], buf.at[pl.ds(0, m)])
        pltpu.sync_copy(buf.at[pl.ds(0, m)], out_hbm.at[pl.ds(out_off + done, m)])
        done += m


def _agg_kernel_body(h_hbm, srcr, dstr, out_hbm,
                     src_v, dst_v, rows_a, rows_b, zbuf, acc_sh, gs):
    c = lax.axis_index("c")
    s = lax.axis_index("s")
    base = s * SUB_ROWS
    node_base = c * HALF

    def zb(i, _):
        for j in range(D // 16):
            zbuf[i, pl.ds(j * 16, 16)] = jnp.zeros((16,), jnp.float32)
        return 0
    lax.fori_loop(0, CHUNK, zb, 0)

    @pl.when(s < NS - 1)
    def _():
        _acc_zero(acc_sh, zbuf, base, SUB_ROWS)

    @pl.when(s == NS - 1)
    def _():
        _acc_zero(acc_sh, zbuf, base, SUB_ROWS_LAST)
        pltpu.sync_copy(zbuf.at[pl.ds(0, ACC_ROWS - HALF)],
                        acc_sh.at[pl.ds(HALF, ACC_ROWS - HALF)])

    pltpu.sync_copy(srcr.at[s], src_v)
    pltpu.sync_copy(dstr.at[s], dst_v)
    _remap_dst(dst_v, node_base)

    plsc.subcore_barrier()

    npair = CHUNKS_PER_TILE // 2
    pltpu.async_copy(h_hbm.at[src_v.at[0]], rows_a, gs)

    def pair_step(jj, _):
        j0 = 2 * jj
        j1 = j0 + 1
        pltpu.make_async_copy(h_hbm.at[src_v.at[j0]], rows_a, gs).wait()
        pltpu.async_copy(h_hbm.at[src_v.at[j1]], rows_b, gs)
        pltpu.sync_copy(rows_a, acc_sh.at[dst_v.at[j0]], add=True)
        pltpu.make_async_copy(h_hbm.at[src_v.at[j1]], rows_b, gs).wait()

        @pl.when(jj < npair - 1)
        def _():
            pltpu.async_copy(h_hbm.at[src_v.at[j0 + 2]], rows_a, gs)

        pltpu.sync_copy(rows_b, acc_sh.at[dst_v.at[j1]], add=True)
        return 0

    lax.fori_loop(0, npair, pair_step, 0)

    plsc.subcore_barrier()

    out_base = node_base + base

    @pl.when(s < NS - 1)
    def _():
        _acc_writeback(acc_sh, rows_a, out_hbm, base, out_base, SUB_ROWS)

    @pl.when(s == NS - 1)
    def _():
        _acc_writeback(acc_sh, rows_a, out_hbm, base, out_base, SUB_ROWS_LAST)


@functools.lru_cache(maxsize=None)
def _make_agg():
    mesh = plsc.VectorSubcoreMesh(core_axis_name="c", subcore_axis_name="s")
    scratch = [
        pltpu.VMEM((CHUNKS_PER_TILE, CHUNK), jnp.int32),
        pltpu.VMEM((CHUNKS_PER_TILE, CHUNK), jnp.int32),
        pltpu.VMEM((CHUNK, D), jnp.float32),
        pltpu.VMEM((CHUNK, D), jnp.float32),
        pltpu.VMEM((CHUNK, D), jnp.float32),
        pltpu.VMEM_SHARED((ACC_ROWS, D), jnp.float32),
        pltpu.SemaphoreType.DMA,
    ]
    return pl.kernel(
        _agg_kernel_body,
        mesh=mesh,
        out_type=jax.ShapeDtypeStruct((N_PAD, D), jnp.float32),
        scratch_types=scratch,
    )


def _cnt_kernel_body(dstr, cnt0_out, cnt1_out, dst_v, ones_v, zcnt, cnt_sh, sem):
    c = lax.axis_index("c")
    s = lax.axis_index("s")
    node_base = c * HALF

    for j in range(CHUNK // 16):
        ones_v[pl.ds(j * 16, 16)] = jnp.ones((16,), jnp.float32)

    def zc(i, _):
        zcnt[pl.ds(i * 16, 16)] = jnp.zeros((16,), jnp.float32)
        return 0
    lax.fori_loop(0, CNT_PAD // 16, zc, 0)

    @pl.when(s == 0)
    def _():
        pltpu.sync_copy(zcnt.at[pl.ds(0, ACC_ROWS)], cnt_sh)

    pltpu.sync_copy(dstr.at[s], dst_v)
    _remap_dst(dst_v, node_base)

    plsc.subcore_barrier()

    fire_k = 8

    def group(g, _):
        for k in range(fire_k):
            pltpu.async_copy(ones_v, cnt_sh.at[dst_v.at[g * fire_k + k]],
                             sem, add=True)
        for k in range(fire_k):
            pltpu.make_async_copy(ones_v, cnt_sh.at[dst_v.at[g * fire_k + k]],
                                  sem).wait()
        return 0
    lax.fori_loop(0, CHUNKS_PER_TILE // fire_k, group, 0)

    plsc.subcore_barrier()

    @pl.when(s == 0)
    def _():
        pltpu.sync_copy(cnt_sh.at[pl.ds(0, HALF)], zcnt.at[pl.ds(0, HALF)])

        @pl.when(c == 0)
        def _():
            pltpu.sync_copy(zcnt, cnt0_out)

        @pl.when(c == 1)
        def _():
            pltpu.sync_copy(zcnt, cnt1_out)


@functools.lru_cache(maxsize=None)
def _make_cnt():
    mesh = plsc.VectorSubcoreMesh(core_axis_name="c", subcore_axis_name="s")
    scratch = [
        pltpu.VMEM((CHUNKS_PER_TILE, CHUNK), jnp.int32),
        pltpu.VMEM((CHUNK,), jnp.float32),
        pltpu.VMEM((CNT_PAD,), jnp.float32),
        pltpu.VMEM_SHARED((ACC_ROWS,), jnp.float32),
        pltpu.SemaphoreType.DMA,
    ]
    return pl.kernel(
        _cnt_kernel_body,
        mesh=mesh,
        out_type=(jax.ShapeDtypeStruct((CNT_PAD,), jnp.float32),
                  jax.ShapeDtypeStruct((CNT_PAD,), jnp.float32)),
        scratch_types=scratch,
    )


def _tc_layer_body(relu, a_ref, cnt_ref, x_ref, wl_ref, wr_ref, b_ref, out_ref):
    rec = 1.0 / jnp.maximum(cnt_ref[...], 1.0)
    mean = a_ref[...] * rec
    z = (jnp.dot(mean, wl_ref[...], preferred_element_type=jnp.float32)
         + jnp.dot(x_ref[...], wr_ref[...], preferred_element_type=jnp.float32)
         + b_ref[...])
    out_ref[...] = jnp.maximum(z, 0.0) if relu else z


@functools.lru_cache(maxsize=None)
def _make_tc_layer(relu):
    B = 1024
    grid = (N_PAD // B,)
    return pl.pallas_call(
        functools.partial(_tc_layer_body, relu),
        grid=grid,
        in_specs=[
            pl.BlockSpec((B, D), lambda i: (i, 0)),
            pl.BlockSpec((B, 1), lambda i: (i, 0)),
            pl.BlockSpec((B, D), lambda i: (i, 0)),
            pl.BlockSpec((D, D), lambda i: (0, 0)),
            pl.BlockSpec((D, D), lambda i: (0, 0)),
            pl.BlockSpec((1, D), lambda i: (0, 0)),
        ],
        out_specs=pl.BlockSpec((B, D), lambda i: (i, 0)),
        out_shape=jax.ShapeDtypeStruct((N_PAD, D), jnp.float32),
    )


def kernel(x, edge_index, W_l1, W_r1, b1, W_l2, W_r2, b2, W_l3, W_r3, b3):
    src = edge_index[0]
    dst = edge_index[1]
    pad_e = E_PAD - E_EDGES
    src_r = jnp.concatenate(
        [src, jnp.zeros((pad_e,), jnp.int32)]).reshape(NS, CHUNKS_PER_TILE, CHUNK)
    dst_r = jnp.concatenate(
        [dst, jnp.full((pad_e,), PAD_DST, jnp.int32)]).reshape(NS, CHUNKS_PER_TILE, CHUNK)

    x_pad = jnp.pad(x, ((0, N_PAD - N_NODES), (0, 0)))

    cnt0, cnt1 = _make_cnt()(dst_r)
    cnt = jnp.concatenate(
        [cnt0[:HALF], cnt1[:HALF],
         jnp.zeros((N_PAD - 2 * HALF,), jnp.float32)]).reshape(N_PAD, 1)

    def pad_w(w):
        return jnp.pad(w, ((0, 0), (0, D - w.shape[1])))

    b1r = b1.reshape(1, D)
    b2r = b2.reshape(1, D)
    b3r = jnp.pad(b3, (0, D - b3.shape[0])).reshape(1, D)

    a1 = _make_agg()(x_pad, src_r, dst_r)
    h1 = _make_tc_layer(True)(a1, cnt, x_pad, W_l1, W_r1, b1r)
    a2 = _make_agg()(h1, src_r, dst_r)
    h2 = _make_tc_layer(True)(a2, cnt, h1, W_l2, W_r2, b2r)
    a3 = _make_agg()(h2, src_r, dst_r)
    out = _make_tc_layer(False)(a3, cnt, h2, pad_w(W_l3), pad_w(W_r3), b3r)
    return out[:N_NODES, :47]


# speedup vs baseline: 1.0199x; 1.0199x over previous
import functools

import jax
import jax.numpy as jnp
from jax import lax
from jax.experimental import pallas as pl
from jax.experimental.pallas import tpu as pltpu
from jax.experimental.pallas import tpu_sc as plsc

NC = 2
NS = 16
NW = NC * NS

N_NODES = 10000
HALF = 5040
ACC_ROWS = HALF + 8
SUB_ROWS = 320
SUB_ROWS_LAST = HALF - (NS - 1) * SUB_ROWS
N_PAD = 10240
E_EDGES = 320000
CHUNK = 128
CHUNKS_PER_TILE = 160
E_PAD = NS * CHUNKS_PER_TILE * CHUNK
PAD_DST = 1 << 20
CNT_PAD = 5120
D = 128


def _remap_dst(dst_v, node_base):
    def remap(i, _):
        for j in range(CHUNK // 16):
            d = dst_v[i, pl.ds(j * 16, 16)] - node_base
            ok = (d >= 0) & (d < HALF)
            dst_v[i, pl.ds(j * 16, 16)] = jnp.where(ok, d, HALF)
        return 0
    lax.fori_loop(0, CHUNKS_PER_TILE, remap, 0)


def _acc_zero(acc_sh, zbuf, off, n):
    done = 0
    while n - done >= CHUNK:
        pltpu.sync_copy(zbuf, acc_sh.at[pl.ds(off + done, CHUNK)])
        done += CHUNK
    if n > done:
        pltpu.sync_copy(zbuf.at[pl.ds(0, n - done)],
                        acc_sh.at[pl.ds(off + done, n - done)])


def _acc_writeback(acc_sh, buf, out_hbm, acc_off, out_off, n):
    done = 0
    while done < n:
        m = min(CHUNK, n - done)
        pltpu.sync_copy(acc_sh.at[pl.ds(acc_off + done, m)], buf.at[pl.ds(0, m)])
        pltpu.sync_copy(buf.at[pl.ds(0, m)], out_hbm.at[pl.ds(out_off + done, m)])
        done += m


def _agg_kernel_body(h_hbm, srcr, dstr, out_hbm,
                     src_v, dst_v, rows_a, rows_b, zbuf, acc_sh, gs):
    c = lax.axis_index("c")
    s = lax.axis_index("s")
    base = s * SUB_ROWS
    node_base = c * HALF

    def zb(i, _):
        for j in range(D // 16):
            zbuf[i, pl.ds(j * 16, 16)] = jnp.zeros((16,), jnp.float32)
        return 0
    lax.fori_loop(0, CHUNK, zb, 0)

    @pl.when(s < NS - 1)
    def _():
        _acc_zero(acc_sh, zbuf, base, SUB_ROWS)

    @pl.when(s == NS - 1)
    def _():
        _acc_zero(acc_sh, zbuf, base, SUB_ROWS_LAST)
        pltpu.sync_copy(zbuf.at[pl.ds(0, ACC_ROWS - HALF)],
                        acc_sh.at[pl.ds(HALF, ACC_ROWS - HALF)])

    pltpu.sync_copy(srcr.at[s], src_v)
    pltpu.sync_copy(dstr.at[s], dst_v)
    _remap_dst(dst_v, node_base)

    plsc.subcore_barrier()

    npair = CHUNKS_PER_TILE // 2
    pltpu.async_copy(h_hbm.at[src_v.at[0]], rows_a, gs)

    def pair_step(jj, _):
        j0 = 2 * jj
        j1 = j0 + 1
        pltpu.make_async_copy(h_hbm.at[src_v.at[j0]], rows_a, gs).wait()
        pltpu.async_copy(h_hbm.at[src_v.at[j1]], rows_b, gs)
        pltpu.make_async_copy(h_hbm.at[src_v.at[j1]], rows_b, gs).wait()

        @pl.when(jj < npair - 1)
        def _():
            pltpu.async_copy(h_hbm.at[src_v.at[j0 + 2]], rows_a, gs)

        return 0

    lax.fori_loop(0, npair, pair_step, 0)

    plsc.subcore_barrier()

    out_base = node_base + base

    @pl.when(s < NS - 1)
    def _():
        _acc_writeback(acc_sh, rows_a, out_hbm, base, out_base, SUB_ROWS)

    @pl.when(s == NS - 1)
    def _():
        _acc_writeback(acc_sh, rows_a, out_hbm, base, out_base, SUB_ROWS_LAST)


@functools.lru_cache(maxsize=None)
def _make_agg():
    mesh = plsc.VectorSubcoreMesh(core_axis_name="c", subcore_axis_name="s")
    scratch = [
        pltpu.VMEM((CHUNKS_PER_TILE, CHUNK), jnp.int32),
        pltpu.VMEM((CHUNKS_PER_TILE, CHUNK), jnp.int32),
        pltpu.VMEM((CHUNK, D), jnp.float32),
        pltpu.VMEM((CHUNK, D), jnp.float32),
        pltpu.VMEM((CHUNK, D), jnp.float32),
        pltpu.VMEM_SHARED((ACC_ROWS, D), jnp.float32),
        pltpu.SemaphoreType.DMA,
    ]
    return pl.kernel(
        _agg_kernel_body,
        mesh=mesh,
        out_type=jax.ShapeDtypeStruct((N_PAD, D), jnp.float32),
        scratch_types=scratch,
    )


def _cnt_kernel_body(dstr, cnt0_out, cnt1_out, dst_v, ones_v, zcnt, cnt_sh, sem):
    c = lax.axis_index("c")
    s = lax.axis_index("s")
    node_base = c * HALF

    for j in range(CHUNK // 16):
        ones_v[pl.ds(j * 16, 16)] = jnp.ones((16,), jnp.float32)

    def zc(i, _):
        zcnt[pl.ds(i * 16, 16)] = jnp.zeros((16,), jnp.float32)
        return 0
    lax.fori_loop(0, CNT_PAD // 16, zc, 0)

    @pl.when(s == 0)
    def _():
        pltpu.sync_copy(zcnt.at[pl.ds(0, ACC_ROWS)], cnt_sh)

    pltpu.sync_copy(dstr.at[s], dst_v)
    _remap_dst(dst_v, node_base)

    plsc.subcore_barrier()

    fire_k = 8

    def group(g, _):
        for k in range(fire_k):
            pltpu.async_copy(ones_v, cnt_sh.at[dst_v.at[g * fire_k + k]],
                             sem, add=True)
        for k in range(fire_k):
            pltpu.make_async_copy(ones_v, cnt_sh.at[dst_v.at[g * fire_k + k]],
                                  sem).wait()
        return 0
    lax.fori_loop(0, CHUNKS_PER_TILE // fire_k, group, 0)

    plsc.subcore_barrier()

    @pl.when(s == 0)
    def _():
        pltpu.sync_copy(cnt_sh.at[pl.ds(0, HALF)], zcnt.at[pl.ds(0, HALF)])

        @pl.when(c == 0)
        def _():
            pltpu.sync_copy(zcnt, cnt0_out)

        @pl.when(c == 1)
        def _():
            pltpu.sync_copy(zcnt, cnt1_out)


@functools.lru_cache(maxsize=None)
def _make_cnt():
    mesh = plsc.VectorSubcoreMesh(core_axis_name="c", subcore_axis_name="s")
    scratch = [
        pltpu.VMEM((CHUNKS_PER_TILE, CHUNK), jnp.int32),
        pltpu.VMEM((CHUNK,), jnp.float32),
        pltpu.VMEM((CNT_PAD,), jnp.float32),
        pltpu.VMEM_SHARED((ACC_ROWS,), jnp.float32),
        pltpu.SemaphoreType.DMA,
    ]
    return pl.kernel(
        _cnt_kernel_body,
        mesh=mesh,
        out_type=(jax.ShapeDtypeStruct((CNT_PAD,), jnp.float32),
                  jax.ShapeDtypeStruct((CNT_PAD,), jnp.float32)),
        scratch_types=scratch,
    )


def _tc_layer_body(relu, a_ref, cnt_ref, x_ref, wl_ref, wr_ref, b_ref, out_ref):
    rec = 1.0 / jnp.maximum(cnt_ref[...], 1.0)
    mean = a_ref[...] * rec
    z = (jnp.dot(mean, wl_ref[...], preferred_element_type=jnp.float32)
         + jnp.dot(x_ref[...], wr_ref[...], preferred_element_type=jnp.float32)
         + b_ref[...])
    out_ref[...] = jnp.maximum(z, 0.0) if relu else z


@functools.lru_cache(maxsize=None)
def _make_tc_layer(relu):
    B = 1024
    grid = (N_PAD // B,)
    return pl.pallas_call(
        functools.partial(_tc_layer_body, relu),
        grid=grid,
        in_specs=[
            pl.BlockSpec((B, D), lambda i: (i, 0)),
            pl.BlockSpec((B, 1), lambda i: (i, 0)),
            pl.BlockSpec((B, D), lambda i: (i, 0)),
            pl.BlockSpec((D, D), lambda i: (0, 0)),
            pl.BlockSpec((D, D), lambda i: (0, 0)),
            pl.BlockSpec((1, D), lambda i: (0, 0)),
        ],
        out_specs=pl.BlockSpec((B, D), lambda i: (i, 0)),
        out_shape=jax.ShapeDtypeStruct((N_PAD, D), jnp.float32),
    )


def kernel(x, edge_index, W_l1, W_r1, b1, W_l2, W_r2, b2, W_l3, W_r3, b3):
    src = edge_index[0]
    dst = edge_index[1]
    pad_e = E_PAD - E_EDGES
    src_r = jnp.concatenate(
        [src, jnp.zeros((pad_e,), jnp.int32)]).reshape(NS, CHUNKS_PER_TILE, CHUNK)
    dst_r = jnp.concatenate(
        [dst, jnp.full((pad_e,), PAD_DST, jnp.int32)]).reshape(NS, CHUNKS_PER_TILE, CHUNK)

    x_pad = jnp.pad(x, ((0, N_PAD - N_NODES), (0, 0)))

    cnt0, cnt1 = _make_cnt()(dst_r)
    cnt = jnp.concatenate(
        [cnt0[:HALF], cnt1[:HALF],
         jnp.zeros((N_PAD - 2 * HALF,), jnp.float32)]).reshape(N_PAD, 1)

    def pad_w(w):
        return jnp.pad(w, ((0, 0), (0, D - w.shape[1])))

    b1r = b1.reshape(1, D)
    b2r = b2.reshape(1, D)
    b3r = jnp.pad(b3, (0, D - b3.shape[0])).reshape(1, D)

    a1 = _make_agg()(x_pad, src_r, dst_r)
    h1 = _make_tc_layer(True)(a1, cnt, x_pad, W_l1, W_r1, b1r)
    a2 = _make_agg()(h1, src_r, dst_r)
    h2 = _make_tc_layer(True)(a2, cnt, h1, W_l2, W_r2, b2r)
    a3 = _make_agg()(h2, src_r, dst_r)
    out = _make_tc_layer(False)(a3, cnt, h2, pad_w(W_l3), pad_w(W_r3), b3r)
    return out[:N_NODES, :47]


# speedup vs baseline: 1.1852x; 1.1620x over previous
import functools

import jax
import jax.numpy as jnp
from jax import lax
from jax.experimental import pallas as pl
from jax.experimental.pallas import tpu as pltpu
from jax.experimental.pallas import tpu_sc as plsc

NC = 2
NS = 16
NW = NC * NS

N_NODES = 10000
HALF = 5040
ACC_ROWS = HALF + 8
SUB_ROWS = 320
SUB_ROWS_LAST = HALF - (NS - 1) * SUB_ROWS
N_PAD = 10240
E_EDGES = 320000
CHUNK = 128
PCHUNKS = 80
E_PAD = NW * PCHUNKS * CHUNK
LIST_CH = PCHUNKS + 1
LIST_LEN = LIST_CH * CHUNK
PAD_DST = 1 << 20
D = 128


def _prefix_sum(scratch, x, iota):
    for sh in (1, 2, 4, 8):
        scratch[...] = x
        sv = plsc.load_gather(scratch, [jnp.maximum(iota - sh, 0)])
        x = x + jnp.where(iota >= sh, sv, 0)
    return x


def _splat_last(scratch, x):
    scratch[...] = x
    return plsc.load_gather(scratch, [jnp.full((16,), 15, jnp.int32)])


def _part_kernel_body(srcr, dstr, esrc_out, edst_out, nch_out,
                      src_v, dst_v, lo_s, lo_d, hi_s, hi_d, cvec,
                      s_sc, dl_sc, dh_sc):
    c = lax.axis_index("c")
    s = lax.axis_index("s")
    p = c * NS + s

    pltpu.sync_copy(srcr.at[p], src_v)
    pltpu.sync_copy(dstr.at[p], dst_v)

    iota = lax.iota(jnp.int32, 16)

    def group(g, offs):
        lo_off, hi_off = offs
        i = g // (CHUNK // 16)
        jj = lax.rem(g, CHUNK // 16)
        sl = pl.ds(jj * 16, 16)
        s16 = src_v[i, sl]
        d16 = dst_v[i, sl]
        m_lo = d16 < HALF
        key = jnp.where(m_lo, jnp.int32(0), jnp.int32(1))
        hi_dloc = d16 - HALF
        d_lo = jnp.where(m_lo, d16, HALF)
        d_hi = jnp.where(
            m_lo, HALF,
            jnp.where((hi_dloc >= 0) & (hi_dloc < HALF), hi_dloc, HALF))
        s_sc[...] = s16
        dl_sc[...] = d_lo
        dh_sc[...] = d_hi
        cvec[...] = jnp.where(m_lo, jnp.int32(1), jnp.int32(0))
        vs = s_sc[...]
        vdl = dl_sc[...]
        vdh = dh_sc[...]
        vm = cvec[...]
        for k in range(16):
            sspl = jnp.full((16,), vs[k], jnp.int32)
            lo_s[pl.ds(lo_off, 16)] = sspl
            lo_d[pl.ds(lo_off, 16)] = jnp.full((16,), vdl[k], jnp.int32)
            hi_s[pl.ds(hi_off, 16)] = sspl
            hi_d[pl.ds(hi_off, 16)] = jnp.full((16,), vdh[k], jnp.int32)
            lo_off = lo_off + vm[k]
            hi_off = hi_off + (1 - vm[k])
        return (lo_off, hi_off)

    lo_cnt, hi_cnt = lax.fori_loop(
        0, PCHUNKS * (CHUNK // 16), group, (jnp.int32(0), jnp.int32(0)))

    zeros16 = jnp.zeros((16,), jnp.int32)
    dummy16 = jnp.full((16,), HALF, jnp.int32)
    for k in range(CHUNK // 16):
        lo_s[pl.ds(lo_cnt + k * 16, 16)] = zeros16
        lo_d[pl.ds(lo_cnt + k * 16, 16)] = dummy16
        hi_s[pl.ds(hi_cnt + k * 16, 16)] = zeros16
        hi_d[pl.ds(hi_cnt + k * 16, 16)] = dummy16

    nch_lo = lax.shift_right_logical(lo_cnt + (CHUNK - 1), 7)
    nch_hi = lax.shift_right_logical(hi_cnt + (CHUNK - 1), 7)
    cvec[...] = jnp.where(iota == 0, nch_lo, jnp.where(iota == 1, nch_hi, 0))

    pltpu.sync_copy(lo_s.at[pl.ds(0, LIST_LEN)], esrc_out.at[0, p])
    pltpu.sync_copy(lo_d.at[pl.ds(0, LIST_LEN)], edst_out.at[0, p])
    pltpu.sync_copy(hi_s.at[pl.ds(0, LIST_LEN)], esrc_out.at[1, p])
    pltpu.sync_copy(hi_d.at[pl.ds(0, LIST_LEN)], edst_out.at[1, p])
    pltpu.sync_copy(cvec, nch_out.at[p])


@functools.lru_cache(maxsize=None)
def _make_part():
    mesh = plsc.VectorSubcoreMesh(core_axis_name="c", subcore_axis_name="s")
    return pl.kernel(
        _part_kernel_body,
        mesh=mesh,
        out_type=(jax.ShapeDtypeStruct((NC, NW, LIST_LEN), jnp.int32),
                  jax.ShapeDtypeStruct((NC, NW, LIST_LEN), jnp.int32),
                  jax.ShapeDtypeStruct((NW, 16), jnp.int32)),
        scratch_types=[
            pltpu.VMEM((PCHUNKS, CHUNK), jnp.int32),
            pltpu.VMEM((PCHUNKS, CHUNK), jnp.int32),
            pltpu.VMEM((LIST_LEN + 16,), jnp.int32),
            pltpu.VMEM((LIST_LEN + 16,), jnp.int32),
            pltpu.VMEM((LIST_LEN + 16,), jnp.int32),
            pltpu.VMEM((LIST_LEN + 16,), jnp.int32),
            pltpu.VMEM((16,), jnp.int32),
            pltpu.VMEM((16,), jnp.int32),
            pltpu.VMEM((16,), jnp.int32),
            pltpu.VMEM((16,), jnp.int32),
        ],
    )


def _read_nch(nch_v, sel):
    v = nch_v[...]
    return jnp.where(sel == 0, v[0], v[1])


def _acc_zero(acc_sh, zbuf, off, n):
    done = 0
    while n - done >= CHUNK:
        pltpu.sync_copy(zbuf, acc_sh.at[pl.ds(off + done, CHUNK)])
        done += CHUNK
    if n > done:
        pltpu.sync_copy(zbuf.at[pl.ds(0, n - done)],
                        acc_sh.at[pl.ds(off + done, n - done)])


def _acc_writeback(acc_sh, buf, out_hbm, acc_off, out_off, n):
    done = 0
    while done < n:
        m = min(CHUNK, n - done)
        pltpu.sync_copy(acc_sh.at[pl.ds(acc_off + done, m)], buf.at[pl.ds(0, m)])
        pltpu.sync_copy(buf.at[pl.ds(0, m)], out_hbm.at[pl.ds(out_off + done, m)])
        done += m


def _agg_kernel_body(h_hbm, esrc, edst, nchr, out_hbm,
                     src_v, dst_v, rows_a, rows_b, nch_v, acc_sh, ga, gb):
    c = lax.axis_index("c")
    s = lax.axis_index("s")
    base = s * SUB_ROWS
    node_base = c * HALF

    zbuf = rows_b

    def zb(i, _):
        for j in range(D // 16):
            zbuf[i, pl.ds(j * 16, 16)] = jnp.zeros((16,), jnp.float32)
        return 0
    lax.fori_loop(0, CHUNK, zb, 0)

    @pl.when(s < NS - 1)
    def _():
        _acc_zero(acc_sh, zbuf, base, SUB_ROWS)

    @pl.when(s == NS - 1)
    def _():
        _acc_zero(acc_sh, zbuf, base, SUB_ROWS_LAST)
        pltpu.sync_copy(zbuf.at[pl.ds(0, ACC_ROWS - HALF)],
                        acc_sh.at[pl.ds(HALF, ACC_ROWS - HALF)])

    plsc.subcore_barrier()

    for k in range(2):
        p = 2 * s + k
        pltpu.sync_copy(nchr.at[p], nch_v)
        nch = _read_nch(nch_v, c)
        pltpu.sync_copy(esrc.at[c, p], src_v)
        pltpu.sync_copy(edst.at[c, p], dst_v)

        @pl.when(nch > 0)
        def _():
            pltpu.async_copy(h_hbm.at[src_v.at[0]], rows_a, ga)

        def pair_step(jj, _):
            j0 = 2 * jj
            j1 = j0 + 1
            w1 = j1 < nch
            w2 = j1 + 1 < nch
            pltpu.make_async_copy(h_hbm.at[src_v.at[j0]], rows_a, ga).wait()

            @pl.when(w1)
            def _():
                pltpu.async_copy(h_hbm.at[src_v.at[j1]], rows_b, gb)

            pltpu.sync_copy(rows_a, acc_sh.at[dst_v.at[j0]], add=True)

            @pl.when(w1)
            def _():
                pltpu.make_async_copy(h_hbm.at[src_v.at[j1]], rows_b, gb).wait()

            @pl.when(w2)
            def _():
                pltpu.async_copy(h_hbm.at[src_v.at[j1 + 1]], rows_a, ga)

            @pl.when(w1)
            def _():
                pltpu.sync_copy(rows_b, acc_sh.at[dst_v.at[j1]], add=True)
            return 0

        lax.fori_loop(0, (nch + 1) // 2, pair_step, 0)

    plsc.subcore_barrier()

    out_base = node_base + base

    @pl.when(s < NS - 1)
    def _():
        _acc_writeback(acc_sh, rows_a, out_hbm, base, out_base, SUB_ROWS)

    @pl.when(s == NS - 1)
    def _():
        _acc_writeback(acc_sh, rows_a, out_hbm, base, out_base, SUB_ROWS_LAST)


@functools.lru_cache(maxsize=None)
def _make_agg():
    mesh = plsc.VectorSubcoreMesh(core_axis_name="c", subcore_axis_name="s")
    scratch = [
        pltpu.VMEM((LIST_CH, CHUNK), jnp.int32),
        pltpu.VMEM((LIST_CH, CHUNK), jnp.int32),
        pltpu.VMEM((CHUNK, D), jnp.float32),
        pltpu.VMEM((CHUNK, D), jnp.float32),
        pltpu.VMEM((16,), jnp.int32),
        pltpu.VMEM_SHARED((ACC_ROWS, D), jnp.float32),
        pltpu.SemaphoreType.DMA,
        pltpu.SemaphoreType.DMA,
    ]
    return pl.kernel(
        _agg_kernel_body,
        mesh=mesh,
        out_type=jax.ShapeDtypeStruct((N_PAD, D), jnp.float32),
        scratch_types=scratch,
    )


def _cnt_kernel_body(edst, nchr, cnt0_out, cnt1_out,
                     dst_v, ones_v, zcnt, nch_v, cnt_sh, sem):
    c = lax.axis_index("c")
    s = lax.axis_index("s")

    for j in range(CHUNK // 16):
        ones_v[pl.ds(j * 16, 16)] = jnp.ones((16,), jnp.float32)

    def zc(i, _):
        zcnt[pl.ds(i * 16, 16)] = jnp.zeros((16,), jnp.float32)
        return 0
    lax.fori_loop(0, (HALF + 80) // 16, zc, 0)

    @pl.when(s == 0)
    def _():
        pltpu.sync_copy(zcnt.at[pl.ds(0, ACC_ROWS)], cnt_sh)

    plsc.subcore_barrier()

    fire_k = 8
    for k in range(2):
        p = 2 * s + k
        pltpu.sync_copy(nchr.at[p], nch_v)
        nch = _read_nch(nch_v, c)
        pltpu.sync_copy(edst.at[c, p], dst_v)

        def group(g, _):
            for kk in range(fire_k):
                @pl.when(g * fire_k + kk < nch)
                def _():
                    pltpu.async_copy(
                        ones_v, cnt_sh.at[dst_v.at[g * fire_k + kk]],
                        sem, add=True)
            for kk in range(fire_k):
                @pl.when(g * fire_k + kk < nch)
                def _():
                    pltpu.make_async_copy(
                        ones_v, cnt_sh.at[dst_v.at[g * fire_k + kk]],
                        sem).wait()
            return 0
        lax.fori_loop(0, (nch + fire_k - 1) // fire_k, group, 0)

    plsc.subcore_barrier()

    @pl.when(s == 0)
    def _():
        pltpu.sync_copy(cnt_sh.at[pl.ds(0, HALF)], zcnt.at[pl.ds(0, HALF)])

        @pl.when(c == 0)
        def _():
            pltpu.sync_copy(zcnt, cnt0_out)

        @pl.when(c == 1)
        def _():
            pltpu.sync_copy(zcnt, cnt1_out)


@functools.lru_cache(maxsize=None)
def _make_cnt():
    mesh = plsc.VectorSubcoreMesh(core_axis_name="c", subcore_axis_name="s")
    scratch = [
        pltpu.VMEM((LIST_CH, CHUNK), jnp.int32),
        pltpu.VMEM((CHUNK,), jnp.float32),
        pltpu.VMEM((HALF + 80,), jnp.float32),
        pltpu.VMEM((16,), jnp.int32),
        pltpu.VMEM_SHARED((ACC_ROWS,), jnp.float32),
        pltpu.SemaphoreType.DMA,
    ]
    return pl.kernel(
        _cnt_kernel_body,
        mesh=mesh,
        out_type=(jax.ShapeDtypeStruct((HALF + 80,), jnp.float32),
                  jax.ShapeDtypeStruct((HALF + 80,), jnp.float32)),
        scratch_types=scratch,
    )


def _tc_layer_body(relu, a_ref, cnt_ref, x_ref, wl_ref, wr_ref, b_ref, out_ref):
    rec = 1.0 / jnp.maximum(cnt_ref[...], 1.0)
    mean = a_ref[...] * rec
    z = (jnp.dot(mean, wl_ref[...], preferred_element_type=jnp.float32)
         + jnp.dot(x_ref[...], wr_ref[...], preferred_element_type=jnp.float32)
         + b_ref[...])
    out_ref[...] = jnp.maximum(z, 0.0) if relu else z


@functools.lru_cache(maxsize=None)
def _make_tc_layer(relu):
    B = 1024
    grid = (N_PAD // B,)
    return pl.pallas_call(
        functools.partial(_tc_layer_body, relu),
        grid=grid,
        in_specs=[
            pl.BlockSpec((B, D), lambda i: (i, 0)),
            pl.BlockSpec((B, 1), lambda i: (i, 0)),
            pl.BlockSpec((B, D), lambda i: (i, 0)),
            pl.BlockSpec((D, D), lambda i: (0, 0)),
            pl.BlockSpec((D, D), lambda i: (0, 0)),
            pl.BlockSpec((1, D), lambda i: (0, 0)),
        ],
        out_specs=pl.BlockSpec((B, D), lambda i: (i, 0)),
        out_shape=jax.ShapeDtypeStruct((N_PAD, D), jnp.float32),
    )


def kernel(x, edge_index, W_l1, W_r1, b1, W_l2, W_r2, b2, W_l3, W_r3, b3):
    src = edge_index[0]
    dst = edge_index[1]
    pad_e = E_PAD - E_EDGES
    src_r = jnp.concatenate(
        [src, jnp.zeros((pad_e,), jnp.int32)]).reshape(NW, PCHUNKS, CHUNK)
    dst_r = jnp.concatenate(
        [dst, jnp.full((pad_e,), PAD_DST, jnp.int32)]).reshape(NW, PCHUNKS, CHUNK)

    x_pad = jnp.pad(x, ((0, N_PAD - N_NODES), (0, 0)))

    esrc1, edst1, nch = _make_part()(src_r, dst_r)
    esrc = esrc1.reshape(NC, NW, LIST_CH, CHUNK)
    edst = edst1.reshape(NC, NW, LIST_CH, CHUNK)

    cnt0, cnt1 = _make_cnt()(edst, nch)
    cnt = jnp.concatenate(
        [cnt0[:HALF], cnt1[:HALF],
         jnp.zeros((N_PAD - 2 * HALF,), jnp.float32)]).reshape(N_PAD, 1)

    def pad_w(w):
        return jnp.pad(w, ((0, 0), (0, D - w.shape[1])))

    b1r = b1.reshape(1, D)
    b2r = b2.reshape(1, D)
    b3r = jnp.pad(b3, (0, D - b3.shape[0])).reshape(1, D)

    agg = _make_agg()
    tc_relu = _make_tc_layer(True)

    a1 = agg(x_pad, esrc, edst, nch)
    h1 = tc_relu(a1, cnt, x_pad, W_l1, W_r1, b1r)
    a2 = agg(h1, esrc, edst, nch)
    h2 = tc_relu(a2, cnt, h1, W_l2, W_r2, b2r)
    a3 = agg(h2, esrc, edst, nch)
    out = _make_tc_layer(False)(a3, cnt, h2, pad_w(W_l3), pad_w(W_r3), b3r)
    return out[:N_NODES, :47]


# speedup vs baseline: 2.8339x; 2.3911x over previous
import functools

import jax
import jax.numpy as jnp
from jax import lax
from jax.experimental import pallas as pl
from jax.experimental.pallas import tpu as pltpu
from jax.experimental.pallas import tpu_sc as plsc

NC = 2
NS = 16
NW = NC * NS

N_NODES = 10000
HALF = 5040
ACC_ROWS = HALF + 8
SUB_ROWS = 320
SUB_ROWS_LAST = HALF - (NS - 1) * SUB_ROWS
N_PAD = 10240
E_EDGES = 320000
CHUNK = 128
PCHUNKS = 80
E_PAD = NW * PCHUNKS * CHUNK
LIST_CH = PCHUNKS + 1
LIST_LEN = LIST_CH * CHUNK
PAD_DST = 1 << 20
D = 128


def _prefix_sum(scratch, x, iota):
    for sh in (1, 2, 4, 8):
        scratch[...] = x
        sv = plsc.load_gather(scratch, [jnp.maximum(iota - sh, 0)])
        x = x + jnp.where(iota >= sh, sv, 0)
    return x


def _splat_last(scratch, x):
    scratch[...] = x
    return plsc.load_gather(scratch, [jnp.full((16,), 15, jnp.int32)])


def _part_kernel_body(srcr, dstr, esrc_out, edst_out, nch_out,
                      src_v, dst_v, lo_s, lo_d, hi_s, hi_d, cvec,
                      s_sc, dl_sc, dh_sc, m2_sc):
    c = lax.axis_index("c")
    s = lax.axis_index("s")
    p = c * NS + s

    pltpu.sync_copy(srcr.at[p], src_v)
    pltpu.sync_copy(dstr.at[p], dst_v)

    iota = lax.iota(jnp.int32, 16)

    def group(g, offs):
        lo_off, hi_off = offs
        i = g // (CHUNK // 16)
        jj = lax.rem(g, CHUNK // 16)
        sl = pl.ds(jj * 16, 16)
        s16 = src_v[i, sl]
        d16 = dst_v[i, sl]
        m_lo = d16 < HALF
        key = jnp.where(m_lo, jnp.int32(0), jnp.int32(1))
        hi_dloc = d16 - HALF
        d_lo = jnp.where(m_lo, d16, HALF)
        d_hi = jnp.where(
            m_lo, HALF,
            jnp.where((hi_dloc >= 0) & (hi_dloc < HALF), hi_dloc, HALF))
        m_hi = (d16 >= HALF) & (d16 < 2 * HALF)
        s_sc[...] = s16
        dl_sc[...] = d_lo
        dh_sc[...] = d_hi
        cvec[...] = jnp.where(m_lo, jnp.int32(1), jnp.int32(0))
        m2_sc[...] = jnp.where(m_hi, jnp.int32(1), jnp.int32(0))
        vs = s_sc[...]
        vdl = dl_sc[...]
        vdh = dh_sc[...]
        vm = cvec[...]
        vm2 = m2_sc[...]
        for k in range(16):
            sspl = jnp.full((16,), vs[k], jnp.int32)
            lo_s[pl.ds(lo_off, 16)] = sspl
            lo_d[pl.ds(lo_off, 16)] = jnp.full((16,), vdl[k], jnp.int32)
            hi_s[pl.ds(hi_off, 16)] = sspl
            hi_d[pl.ds(hi_off, 16)] = jnp.full((16,), vdh[k], jnp.int32)
            lo_off = lo_off + vm[k]
            hi_off = hi_off + vm2[k]
        return (lo_off, hi_off)

    lo_cnt, hi_cnt = lax.fori_loop(
        0, PCHUNKS * (CHUNK // 16), group, (jnp.int32(0), jnp.int32(0)))

    zeros16 = jnp.zeros((16,), jnp.int32)
    dummy16 = jnp.full((16,), HALF, jnp.int32)
    for k in range(CHUNK // 16):
        lo_s[pl.ds(lo_cnt + k * 16, 16)] = zeros16
        lo_d[pl.ds(lo_cnt + k * 16, 16)] = dummy16
        hi_s[pl.ds(hi_cnt + k * 16, 16)] = zeros16
        hi_d[pl.ds(hi_cnt + k * 16, 16)] = dummy16

    nch_lo = lax.shift_right_logical(lo_cnt + (CHUNK - 1), 7)
    nch_hi = lax.shift_right_logical(hi_cnt + (CHUNK - 1), 7)
    cvec[...] = jnp.where(iota == 0, nch_lo, jnp.where(iota == 1, nch_hi, 0))

    pltpu.sync_copy(lo_s.at[pl.ds(0, LIST_LEN)], esrc_out.at[0, p])
    pltpu.sync_copy(lo_d.at[pl.ds(0, LIST_LEN)], edst_out.at[0, p])
    pltpu.sync_copy(hi_s.at[pl.ds(0, LIST_LEN)], esrc_out.at[1, p])
    pltpu.sync_copy(hi_d.at[pl.ds(0, LIST_LEN)], edst_out.at[1, p])
    pltpu.sync_copy(cvec, nch_out.at[p])


@functools.lru_cache(maxsize=None)
def _make_part():
    mesh = plsc.VectorSubcoreMesh(core_axis_name="c", subcore_axis_name="s")
    return pl.kernel(
        _part_kernel_body,
        mesh=mesh,
        out_type=(jax.ShapeDtypeStruct((NC, NW, LIST_LEN), jnp.int32),
                  jax.ShapeDtypeStruct((NC, NW, LIST_LEN), jnp.int32),
                  jax.ShapeDtypeStruct((NW, 16), jnp.int32)),
        scratch_types=[
            pltpu.VMEM((PCHUNKS, CHUNK), jnp.int32),
            pltpu.VMEM((PCHUNKS, CHUNK), jnp.int32),
            pltpu.VMEM((LIST_LEN + 16,), jnp.int32),
            pltpu.VMEM((LIST_LEN + 16,), jnp.int32),
            pltpu.VMEM((LIST_LEN + 16,), jnp.int32),
            pltpu.VMEM((LIST_LEN + 16,), jnp.int32),
            pltpu.VMEM((16,), jnp.int32),
            pltpu.VMEM((16,), jnp.int32),
            pltpu.VMEM((16,), jnp.int32),
            pltpu.VMEM((16,), jnp.int32),
            pltpu.VMEM((16,), jnp.int32),
        ],
    )


def _read_nch(nch_v, sel):
    v = nch_v[...]
    return jnp.where(sel == 0, v[0], v[1])


def _acc_zero(acc_sh, zbuf, off, n):
    done = 0
    while n - done >= CHUNK:
        pltpu.sync_copy(zbuf, acc_sh.at[pl.ds(off + done, CHUNK)])
        done += CHUNK
    if n > done:
        pltpu.sync_copy(zbuf.at[pl.ds(0, n - done)],
                        acc_sh.at[pl.ds(off + done, n - done)])


def _acc_writeback(acc_sh, buf, out_hbm, acc_off, out_off, n):
    done = 0
    while done < n:
        m = min(CHUNK, n - done)
        pltpu.sync_copy(acc_sh.at[pl.ds(acc_off + done, m)], buf.at[pl.ds(0, m)])
        pltpu.sync_copy(buf.at[pl.ds(0, m)], out_hbm.at[pl.ds(out_off + done, m)])
        done += m


def _agg_kernel_body(h_hbm, esrc, edst, nchr, out_hbm,
                     src_v, dst_v, rows_a, rows_b, nch_v, acc_sh, ga, gb):
    c = lax.axis_index("c")
    s = lax.axis_index("s")
    base = s * SUB_ROWS
    node_base = c * HALF

    zbuf = rows_b

    def zb(i, _):
        for j in range(D // 16):
            zbuf[i, pl.ds(j * 16, 16)] = jnp.zeros((16,), jnp.float32)
        return 0
    lax.fori_loop(0, CHUNK, zb, 0)

    @pl.when(s < NS - 1)
    def _():
        _acc_zero(acc_sh, zbuf, base, SUB_ROWS)

    @pl.when(s == NS - 1)
    def _():
        _acc_zero(acc_sh, zbuf, base, SUB_ROWS_LAST)
        pltpu.sync_copy(zbuf.at[pl.ds(0, ACC_ROWS - HALF)],
                        acc_sh.at[pl.ds(HALF, ACC_ROWS - HALF)])

    plsc.subcore_barrier()

    for k in range(2):
        p = 2 * s + k
        pltpu.sync_copy(nchr.at[p], nch_v)
        nch = _read_nch(nch_v, c)
        pltpu.sync_copy(esrc.at[c, p], src_v)
        pltpu.sync_copy(edst.at[c, p], dst_v)

        @pl.when(nch > 0)
        def _():
            pltpu.async_copy(h_hbm.at[src_v.at[0]], rows_a, ga)

        def pair_step(jj, _):
            j0 = 2 * jj
            j1 = j0 + 1
            w1 = j1 < nch
            w2 = j1 + 1 < nch
            pltpu.make_async_copy(h_hbm.at[src_v.at[j0]], rows_a, ga).wait()

            @pl.when(w1)
            def _():
                pltpu.async_copy(h_hbm.at[src_v.at[j1]], rows_b, gb)

            pltpu.sync_copy(rows_a, acc_sh.at[dst_v.at[j0]], add=True)

            @pl.when(w1)
            def _():
                pltpu.make_async_copy(h_hbm.at[src_v.at[j1]], rows_b, gb).wait()

            @pl.when(w2)
            def _():
                pltpu.async_copy(h_hbm.at[src_v.at[j1 + 1]], rows_a, ga)

            @pl.when(w1)
            def _():
                pltpu.sync_copy(rows_b, acc_sh.at[dst_v.at[j1]], add=True)
            return 0

        lax.fori_loop(0, (nch + 1) // 2, pair_step, 0)

    plsc.subcore_barrier()

    out_base = node_base + base

    @pl.when(s < NS - 1)
    def _():
        _acc_writeback(acc_sh, rows_a, out_hbm, base, out_base, SUB_ROWS)

    @pl.when(s == NS - 1)
    def _():
        _acc_writeback(acc_sh, rows_a, out_hbm, base, out_base, SUB_ROWS_LAST)


@functools.lru_cache(maxsize=None)
def _make_agg():
    mesh = plsc.VectorSubcoreMesh(core_axis_name="c", subcore_axis_name="s")
    scratch = [
        pltpu.VMEM((LIST_CH, CHUNK), jnp.int32),
        pltpu.VMEM((LIST_CH, CHUNK), jnp.int32),
        pltpu.VMEM((CHUNK, D), jnp.float32),
        pltpu.VMEM((CHUNK, D), jnp.float32),
        pltpu.VMEM((16,), jnp.int32),
        pltpu.VMEM_SHARED((ACC_ROWS, D), jnp.float32),
        pltpu.SemaphoreType.DMA,
        pltpu.SemaphoreType.DMA,
    ]
    return pl.kernel(
        _agg_kernel_body,
        mesh=mesh,
        out_type=jax.ShapeDtypeStruct((N_PAD, D), jnp.float32),
        scratch_types=scratch,
    )


def _cnt_kernel_body(edst, nchr, cnt0_out, cnt1_out,
                     dst_v, ones_v, zcnt, nch_v, cnt_sh, sem):
    c = lax.axis_index("c")
    s = lax.axis_index("s")

    for j in range(CHUNK // 16):
        ones_v[pl.ds(j * 16, 16)] = jnp.ones((16,), jnp.float32)

    def zc(i, _):
        zcnt[pl.ds(i * 16, 16)] = jnp.zeros((16,), jnp.float32)
        return 0
    lax.fori_loop(0, (HALF + 80) // 16, zc, 0)

    @pl.when(s == 0)
    def _():
        pltpu.sync_copy(zcnt.at[pl.ds(0, ACC_ROWS)], cnt_sh)

    plsc.subcore_barrier()

    fire_k = 8
    for k in range(2):
        p = 2 * s + k
        pltpu.sync_copy(nchr.at[p], nch_v)
        nch = _read_nch(nch_v, c)
        pltpu.sync_copy(edst.at[c, p], dst_v)

        def group(g, _):
            for kk in range(fire_k):
                @pl.when(g * fire_k + kk < nch)
                def _():
                    pltpu.async_copy(
                        ones_v, cnt_sh.at[dst_v.at[g * fire_k + kk]],
                        sem, add=True)
            for kk in range(fire_k):
                @pl.when(g * fire_k + kk < nch)
                def _():
                    pltpu.make_async_copy(
                        ones_v, cnt_sh.at[dst_v.at[g * fire_k + kk]],
                        sem).wait()
            return 0
        lax.fori_loop(0, (nch + fire_k - 1) // fire_k, group, 0)

    plsc.subcore_barrier()

    @pl.when(s == 0)
    def _():
        pltpu.sync_copy(cnt_sh.at[pl.ds(0, HALF)], zcnt.at[pl.ds(0, HALF)])

        @pl.when(c == 0)
        def _():
            pltpu.sync_copy(zcnt, cnt0_out)

        @pl.when(c == 1)
        def _():
            pltpu.sync_copy(zcnt, cnt1_out)


@functools.lru_cache(maxsize=None)
def _make_cnt():
    mesh = plsc.VectorSubcoreMesh(core_axis_name="c", subcore_axis_name="s")
    scratch = [
        pltpu.VMEM((LIST_CH, CHUNK), jnp.int32),
        pltpu.VMEM((CHUNK,), jnp.float32),
        pltpu.VMEM((HALF + 80,), jnp.float32),
        pltpu.VMEM((16,), jnp.int32),
        pltpu.VMEM_SHARED((ACC_ROWS,), jnp.float32),
        pltpu.SemaphoreType.DMA,
    ]
    return pl.kernel(
        _cnt_kernel_body,
        mesh=mesh,
        out_type=(jax.ShapeDtypeStruct((HALF + 80,), jnp.float32),
                  jax.ShapeDtypeStruct((HALF + 80,), jnp.float32)),
        scratch_types=scratch,
    )


def _tc_layer_body(relu, a_ref, cnt_ref, x_ref, wl_ref, wr_ref, b_ref, out_ref):
    rec = 1.0 / jnp.maximum(cnt_ref[...], 1.0)
    mean = a_ref[...] * rec
    z = (jnp.dot(mean, wl_ref[...], preferred_element_type=jnp.float32)
         + jnp.dot(x_ref[...], wr_ref[...], preferred_element_type=jnp.float32)
         + b_ref[...])
    out_ref[...] = jnp.maximum(z, 0.0) if relu else z


@functools.lru_cache(maxsize=None)
def _make_tc_layer(relu):
    B = 1024
    grid = (N_PAD // B,)
    return pl.pallas_call(
        functools.partial(_tc_layer_body, relu),
        grid=grid,
        in_specs=[
            pl.BlockSpec((B, D), lambda i: (i, 0)),
            pl.BlockSpec((B, 1), lambda i: (i, 0)),
            pl.BlockSpec((B, D), lambda i: (i, 0)),
            pl.BlockSpec((D, D), lambda i: (0, 0)),
            pl.BlockSpec((D, D), lambda i: (0, 0)),
            pl.BlockSpec((1, D), lambda i: (0, 0)),
        ],
        out_specs=pl.BlockSpec((B, D), lambda i: (i, 0)),
        out_shape=jax.ShapeDtypeStruct((N_PAD, D), jnp.float32),
    )


def kernel(x, edge_index, W_l1, W_r1, b1, W_l2, W_r2, b2, W_l3, W_r3, b3):
    src = edge_index[0]
    dst = edge_index[1]
    pad_e = E_PAD - E_EDGES
    src_r = jnp.concatenate(
        [src, jnp.zeros((pad_e,), jnp.int32)]).reshape(NW, PCHUNKS, CHUNK)
    dst_r = jnp.concatenate(
        [dst, jnp.full((pad_e,), PAD_DST, jnp.int32)]).reshape(NW, PCHUNKS, CHUNK)

    x_pad = jnp.pad(x, ((0, N_PAD - N_NODES), (0, 0)))

    esrc1, edst1, nch = _make_part()(src_r, dst_r)
    esrc = esrc1.reshape(NC, NW, LIST_CH, CHUNK)
    edst = edst1.reshape(NC, NW, LIST_CH, CHUNK)

    cnt0, cnt1 = _make_cnt()(edst, nch)
    cnt = jnp.concatenate(
        [cnt0[:HALF], cnt1[:HALF],
         jnp.zeros((N_PAD - 2 * HALF,), jnp.float32)]).reshape(N_PAD, 1)

    def pad_w(w):
        return jnp.pad(w, ((0, 0), (0, D - w.shape[1])))

    b1r = b1.reshape(1, D)
    b2r = b2.reshape(1, D)
    b3r = jnp.pad(b3, (0, D - b3.shape[0])).reshape(1, D)

    agg = _make_agg()
    tc_relu = _make_tc_layer(True)

    a1 = agg(x_pad, esrc, edst, nch)
    h1 = tc_relu(a1, cnt, x_pad, W_l1, W_r1, b1r)
    a2 = agg(h1, esrc, edst, nch)
    h2 = tc_relu(a2, cnt, h1, W_l2, W_r2, b2r)
    a3 = agg(h2, esrc, edst, nch)
    out = _make_tc_layer(False)(a3, cnt, h2, pad_w(W_l3), pad_w(W_r3), b3r)
    return out[:N_NODES, :47]


# speedup vs baseline: 2.8688x; 1.0123x over previous
import functools

import jax
import jax.numpy as jnp
from jax import lax
from jax.experimental import pallas as pl
from jax.experimental.pallas import tpu as pltpu
from jax.experimental.pallas import tpu_sc as plsc

NC = 2
NS = 16
NW = NC * NS

N_NODES = 10000
HALF = 5040
ACC_ROWS = HALF + 8
SUB_ROWS = 320
SUB_ROWS_LAST = HALF - (NS - 1) * SUB_ROWS
N_PAD = 10240
E_EDGES = 320000
CHUNK = 128
PCHUNKS = 80
E_PAD = NW * PCHUNKS * CHUNK
LIST_CH = PCHUNKS + 1
LIST_LEN = LIST_CH * CHUNK
PAD_DST = 1 << 20
D = 128


def _prefix_sum(scratch, x, iota):
    for sh in (1, 2, 4, 8):
        scratch[...] = x
        sv = plsc.load_gather(scratch, [jnp.maximum(iota - sh, 0)])
        x = x + jnp.where(iota >= sh, sv, 0)
    return x


def _splat_last(scratch, x):
    scratch[...] = x
    return plsc.load_gather(scratch, [jnp.full((16,), 15, jnp.int32)])


def _part_kernel_body(srcr, dstr, esrc_out, edst_out, nch_out,
                      src_v, dst_v, lo_s, lo_d, hi_s, hi_d, cvec,
                      s_sc, dl_sc, dh_sc, m2_sc):
    c = lax.axis_index("c")
    s = lax.axis_index("s")
    p = c * NS + s

    pltpu.sync_copy(srcr.at[p], src_v)
    pltpu.sync_copy(dstr.at[p], dst_v)

    iota = lax.iota(jnp.int32, 16)

    def group(g, offs):
        lo_off, hi_off = offs
        i = g // (CHUNK // 16)
        jj = lax.rem(g, CHUNK // 16)
        sl = pl.ds(jj * 16, 16)
        s16 = src_v[i, sl]
        d16 = dst_v[i, sl]
        m_lo = d16 < HALF
        key = jnp.where(m_lo, jnp.int32(0), jnp.int32(1))
        hi_dloc = d16 - HALF
        d_lo = jnp.where(m_lo, d16, HALF)
        d_hi = jnp.where(
            m_lo, HALF,
            jnp.where((hi_dloc >= 0) & (hi_dloc < HALF), hi_dloc, HALF))
        m_hi = (d16 >= HALF) & (d16 < 2 * HALF)
        s_sc[...] = s16
        dl_sc[...] = d_lo
        dh_sc[...] = d_hi
        cvec[...] = jnp.where(m_lo, jnp.int32(1), jnp.int32(0))
        m2_sc[...] = jnp.where(m_hi, jnp.int32(1), jnp.int32(0))
        vs = s_sc[...]
        vdl = dl_sc[...]
        vdh = dh_sc[...]
        vm = cvec[...]
        vm2 = m2_sc[...]
        for k in range(16):
            sspl = jnp.full((16,), vs[k], jnp.int32)
            lo_s[pl.ds(lo_off, 16)] = sspl
            lo_d[pl.ds(lo_off, 16)] = jnp.full((16,), vdl[k], jnp.int32)
            hi_s[pl.ds(hi_off, 16)] = sspl
            hi_d[pl.ds(hi_off, 16)] = jnp.full((16,), vdh[k], jnp.int32)
            lo_off = lo_off + vm[k]
            hi_off = hi_off + vm2[k]
        return (lo_off, hi_off)

    lo_cnt, hi_cnt = lax.fori_loop(
        0, PCHUNKS * (CHUNK // 16), group, (jnp.int32(0), jnp.int32(0)))

    zeros16 = jnp.zeros((16,), jnp.int32)
    dummy16 = jnp.full((16,), HALF, jnp.int32)
    for k in range(CHUNK // 16):
        lo_s[pl.ds(lo_cnt + k * 16, 16)] = zeros16
        lo_d[pl.ds(lo_cnt + k * 16, 16)] = dummy16
        hi_s[pl.ds(hi_cnt + k * 16, 16)] = zeros16
        hi_d[pl.ds(hi_cnt + k * 16, 16)] = dummy16

    nch_lo = lax.shift_right_logical(lo_cnt + (CHUNK - 1), 7)
    nch_hi = lax.shift_right_logical(hi_cnt + (CHUNK - 1), 7)
    cvec[...] = jnp.where(iota == 0, nch_lo, jnp.where(iota == 1, nch_hi, 0))

    pltpu.sync_copy(lo_s.at[pl.ds(0, LIST_LEN)], esrc_out.at[0, p])
    pltpu.sync_copy(lo_d.at[pl.ds(0, LIST_LEN)], edst_out.at[0, p])
    pltpu.sync_copy(hi_s.at[pl.ds(0, LIST_LEN)], esrc_out.at[1, p])
    pltpu.sync_copy(hi_d.at[pl.ds(0, LIST_LEN)], edst_out.at[1, p])
    pltpu.sync_copy(cvec, nch_out.at[p])


@functools.lru_cache(maxsize=None)
def _make_part():
    mesh = plsc.VectorSubcoreMesh(core_axis_name="c", subcore_axis_name="s")
    return pl.kernel(
        _part_kernel_body,
        mesh=mesh,
        out_type=(jax.ShapeDtypeStruct((NC, NW, LIST_LEN), jnp.int32),
                  jax.ShapeDtypeStruct((NC, NW, LIST_LEN), jnp.int32),
                  jax.ShapeDtypeStruct((NW, 16), jnp.int32)),
        scratch_types=[
            pltpu.VMEM((PCHUNKS, CHUNK), jnp.int32),
            pltpu.VMEM((PCHUNKS, CHUNK), jnp.int32),
            pltpu.VMEM((LIST_LEN + 16,), jnp.int32),
            pltpu.VMEM((LIST_LEN + 16,), jnp.int32),
            pltpu.VMEM((LIST_LEN + 16,), jnp.int32),
            pltpu.VMEM((LIST_LEN + 16,), jnp.int32),
            pltpu.VMEM((16,), jnp.int32),
            pltpu.VMEM((16,), jnp.int32),
            pltpu.VMEM((16,), jnp.int32),
            pltpu.VMEM((16,), jnp.int32),
            pltpu.VMEM((16,), jnp.int32),
        ],
    )


def _read_nch(nch_v, sel):
    v = nch_v[...]
    return jnp.where(sel == 0, v[0], v[1])


def _acc_zero(acc_sh, zbuf, off, n):
    done = 0
    while n - done >= CHUNK:
        pltpu.sync_copy(zbuf, acc_sh.at[pl.ds(off + done, CHUNK)])
        done += CHUNK
    if n > done:
        pltpu.sync_copy(zbuf.at[pl.ds(0, n - done)],
                        acc_sh.at[pl.ds(off + done, n - done)])


def _acc_writeback(acc_sh, buf, out_hbm, acc_off, out_off, n):
    done = 0
    while done < n:
        m = min(CHUNK, n - done)
        pltpu.sync_copy(acc_sh.at[pl.ds(acc_off + done, m)], buf.at[pl.ds(0, m)])
        pltpu.sync_copy(buf.at[pl.ds(0, m)], out_hbm.at[pl.ds(out_off + done, m)])
        done += m


def _agg_kernel_body(h_hbm, esrc, edst, nchr, out_hbm,
                     src_v, dst_v, rows_a, rows_b, nch_v, acc_sh, ga, gb):
    c = lax.axis_index("c")
    s = lax.axis_index("s")
    base = s * SUB_ROWS
    node_base = c * HALF

    zbuf = rows_b

    def zb(i, _):
        for j in range(D // 16):
            zbuf[i, pl.ds(j * 16, 16)] = jnp.zeros((16,), jnp.float32)
        return 0
    lax.fori_loop(0, CHUNK, zb, 0)

    @pl.when(s < NS - 1)
    def _():
        _acc_zero(acc_sh, zbuf, base, SUB_ROWS)

    @pl.when(s == NS - 1)
    def _():
        _acc_zero(acc_sh, zbuf, base, SUB_ROWS_LAST)
        pltpu.sync_copy(zbuf.at[pl.ds(0, ACC_ROWS - HALF)],
                        acc_sh.at[pl.ds(HALF, ACC_ROWS - HALF)])

    plsc.subcore_barrier()

    for k in range(2):
        p = 2 * s + k
        pltpu.sync_copy(nchr.at[p], nch_v)
        nch = _read_nch(nch_v, c)
        pltpu.sync_copy(esrc.at[c, p], src_v)
        pltpu.sync_copy(edst.at[c, p], dst_v)

        @pl.when(nch > 0)
        def _():
            pltpu.async_copy(h_hbm.at[src_v.at[0]], rows_a, ga)

        def pair_step(jj, _):
            j0 = 2 * jj
            j1 = j0 + 1
            w1 = j1 < nch
            w2 = j1 + 1 < nch
            pltpu.make_async_copy(h_hbm.at[src_v.at[j0]], rows_a, ga).wait()

            @pl.when(w1)
            def _():
                pltpu.async_copy(h_hbm.at[src_v.at[j1]], rows_b, gb)

            pass

            @pl.when(w1)
            def _():
                pltpu.make_async_copy(h_hbm.at[src_v.at[j1]], rows_b, gb).wait()

            @pl.when(w2)
            def _():
                pltpu.async_copy(h_hbm.at[src_v.at[j1 + 1]], rows_a, ga)

            pass
            return 0

        lax.fori_loop(0, (nch + 1) // 2, pair_step, 0)

    plsc.subcore_barrier()

    out_base = node_base + base

    @pl.when(s < NS - 1)
    def _():
        _acc_writeback(acc_sh, rows_a, out_hbm, base, out_base, SUB_ROWS)

    @pl.when(s == NS - 1)
    def _():
        _acc_writeback(acc_sh, rows_a, out_hbm, base, out_base, SUB_ROWS_LAST)


@functools.lru_cache(maxsize=None)
def _make_agg():
    mesh = plsc.VectorSubcoreMesh(core_axis_name="c", subcore_axis_name="s")
    scratch = [
        pltpu.VMEM((LIST_CH, CHUNK), jnp.int32),
        pltpu.VMEM((LIST_CH, CHUNK), jnp.int32),
        pltpu.VMEM((CHUNK, D), jnp.float32),
        pltpu.VMEM((CHUNK, D), jnp.float32),
        pltpu.VMEM((16,), jnp.int32),
        pltpu.VMEM_SHARED((ACC_ROWS, D), jnp.float32),
        pltpu.SemaphoreType.DMA,
        pltpu.SemaphoreType.DMA,
    ]
    return pl.kernel(
        _agg_kernel_body,
        mesh=mesh,
        out_type=jax.ShapeDtypeStruct((N_PAD, D), jnp.float32),
        scratch_types=scratch,
    )


def _cnt_kernel_body(edst, nchr, cnt0_out, cnt1_out,
                     dst_v, ones_v, zcnt, nch_v, cnt_sh, sem):
    c = lax.axis_index("c")
    s = lax.axis_index("s")

    for j in range(CHUNK // 16):
        ones_v[pl.ds(j * 16, 16)] = jnp.ones((16,), jnp.float32)

    def zc(i, _):
        zcnt[pl.ds(i * 16, 16)] = jnp.zeros((16,), jnp.float32)
        return 0
    lax.fori_loop(0, (HALF + 80) // 16, zc, 0)

    @pl.when(s == 0)
    def _():
        pltpu.sync_copy(zcnt.at[pl.ds(0, ACC_ROWS)], cnt_sh)

    plsc.subcore_barrier()

    fire_k = 8
    for k in range(2):
        p = 2 * s + k
        pltpu.sync_copy(nchr.at[p], nch_v)
        nch = _read_nch(nch_v, c)
        pltpu.sync_copy(edst.at[c, p], dst_v)

        def group(g, _):
            for kk in range(fire_k):
                @pl.when(g * fire_k + kk < nch)
                def _():
                    pltpu.async_copy(
                        ones_v, cnt_sh.at[dst_v.at[g * fire_k + kk]],
                        sem, add=True)
            for kk in range(fire_k):
                @pl.when(g * fire_k + kk < nch)
                def _():
                    pltpu.make_async_copy(
                        ones_v, cnt_sh.at[dst_v.at[g * fire_k + kk]],
                        sem).wait()
            return 0
        lax.fori_loop(0, (nch + fire_k - 1) // fire_k, group, 0)

    plsc.subcore_barrier()

    @pl.when(s == 0)
    def _():
        pltpu.sync_copy(cnt_sh.at[pl.ds(0, HALF)], zcnt.at[pl.ds(0, HALF)])

        @pl.when(c == 0)
        def _():
            pltpu.sync_copy(zcnt, cnt0_out)

        @pl.when(c == 1)
        def _():
            pltpu.sync_copy(zcnt, cnt1_out)


@functools.lru_cache(maxsize=None)
def _make_cnt():
    mesh = plsc.VectorSubcoreMesh(core_axis_name="c", subcore_axis_name="s")
    scratch = [
        pltpu.VMEM((LIST_CH, CHUNK), jnp.int32),
        pltpu.VMEM((CHUNK,), jnp.float32),
        pltpu.VMEM((HALF + 80,), jnp.float32),
        pltpu.VMEM((16,), jnp.int32),
        pltpu.VMEM_SHARED((ACC_ROWS,), jnp.float32),
        pltpu.SemaphoreType.DMA,
    ]
    return pl.kernel(
        _cnt_kernel_body,
        mesh=mesh,
        out_type=(jax.ShapeDtypeStruct((HALF + 80,), jnp.float32),
                  jax.ShapeDtypeStruct((HALF + 80,), jnp.float32)),
        scratch_types=scratch,
    )


def _tc_layer_body(relu, a_ref, cnt_ref, x_ref, wl_ref, wr_ref, b_ref, out_ref):
    rec = 1.0 / jnp.maximum(cnt_ref[...], 1.0)
    mean = a_ref[...] * rec
    z = (jnp.dot(mean, wl_ref[...], preferred_element_type=jnp.float32)
         + jnp.dot(x_ref[...], wr_ref[...], preferred_element_type=jnp.float32)
         + b_ref[...])
    out_ref[...] = jnp.maximum(z, 0.0) if relu else z


@functools.lru_cache(maxsize=None)
def _make_tc_layer(relu):
    B = 1024
    grid = (N_PAD // B,)
    return pl.pallas_call(
        functools.partial(_tc_layer_body, relu),
        grid=grid,
        in_specs=[
            pl.BlockSpec((B, D), lambda i: (i, 0)),
            pl.BlockSpec((B, 1), lambda i: (i, 0)),
            pl.BlockSpec((B, D), lambda i: (i, 0)),
            pl.BlockSpec((D, D), lambda i: (0, 0)),
            pl.BlockSpec((D, D), lambda i: (0, 0)),
            pl.BlockSpec((1, D), lambda i: (0, 0)),
        ],
        out_specs=pl.BlockSpec((B, D), lambda i: (i, 0)),
        out_shape=jax.ShapeDtypeStruct((N_PAD, D), jnp.float32),
    )


def kernel(x, edge_index, W_l1, W_r1, b1, W_l2, W_r2, b2, W_l3, W_r3, b3):
    src = edge_index[0]
    dst = edge_index[1]
    pad_e = E_PAD - E_EDGES
    src_r = jnp.concatenate(
        [src, jnp.zeros((pad_e,), jnp.int32)]).reshape(NW, PCHUNKS, CHUNK)
    dst_r = jnp.concatenate(
        [dst, jnp.full((pad_e,), PAD_DST, jnp.int32)]).reshape(NW, PCHUNKS, CHUNK)

    x_pad = jnp.pad(x, ((0, N_PAD - N_NODES), (0, 0)))

    esrc1, edst1, nch = _make_part()(src_r, dst_r)
    esrc = esrc1.reshape(NC, NW, LIST_CH, CHUNK)
    edst = edst1.reshape(NC, NW, LIST_CH, CHUNK)

    cnt0, cnt1 = _make_cnt()(edst, nch)
    cnt = jnp.concatenate(
        [cnt0[:HALF], cnt1[:HALF],
         jnp.zeros((N_PAD - 2 * HALF,), jnp.float32)]).reshape(N_PAD, 1)

    def pad_w(w):
        return jnp.pad(w, ((0, 0), (0, D - w.shape[1])))

    b1r = b1.reshape(1, D)
    b2r = b2.reshape(1, D)
    b3r = jnp.pad(b3, (0, D - b3.shape[0])).reshape(1, D)

    agg = _make_agg()
    tc_relu = _make_tc_layer(True)

    a1 = agg(x_pad, esrc, edst, nch)
    h1 = tc_relu(a1, cnt, x_pad, W_l1, W_r1, b1r)
    a2 = agg(h1, esrc, edst, nch)
    h2 = tc_relu(a2, cnt, h1, W_l2, W_r2, b2r)
    a3 = agg(h2, esrc, edst, nch)
    out = _make_tc_layer(False)(a3, cnt, h2, pad_w(W_l3), pad_w(W_r3), b3r)
    return out[:N_NODES, :47]


# speedup vs baseline: 3.1318x; 1.0917x over previous
import functools

import jax
import jax.numpy as jnp
from jax import lax
from jax.experimental import pallas as pl
from jax.experimental.pallas import tpu as pltpu
from jax.experimental.pallas import tpu_sc as plsc

NC = 2
NS = 16
NW = NC * NS

N_NODES = 10000
HALF = 5040
ACC_ROWS = HALF + 8
SUB_ROWS = 320
SUB_ROWS_LAST = HALF - (NS - 1) * SUB_ROWS
N_PAD = 10240
E_EDGES = 320000
CHUNK = 128
PCHUNKS = 80
E_PAD = NW * PCHUNKS * CHUNK
LIST_CH = PCHUNKS + 1
LIST_LEN = LIST_CH * CHUNK
PAD_DST = 1 << 20
D = 128


def _prefix_sum(scratch, x, iota):
    for sh in (1, 2, 4, 8):
        scratch[...] = x
        sv = plsc.load_gather(scratch, [jnp.maximum(iota - sh, 0)])
        x = x + jnp.where(iota >= sh, sv, 0)
    return x


def _splat_last(scratch, x):
    scratch[...] = x
    return plsc.load_gather(scratch, [jnp.full((16,), 15, jnp.int32)])


def _part_kernel_body(srcr, dstr, esrc_out, edst_out, nch_out,
                      src_v, dst_v, lo_s, lo_d, hi_s, hi_d, cvec,
                      s_sc, dl_sc, dh_sc, m2_sc):
    c = lax.axis_index("c")
    s = lax.axis_index("s")
    p = c * NS + s

    pltpu.sync_copy(srcr.at[p], src_v)
    pltpu.sync_copy(dstr.at[p], dst_v)

    iota = lax.iota(jnp.int32, 16)

    def group(g, offs):
        lo_off, hi_off = offs
        i = g // (CHUNK // 16)
        jj = lax.rem(g, CHUNK // 16)
        sl = pl.ds(jj * 16, 16)
        s16 = src_v[i, sl]
        d16 = dst_v[i, sl]
        m_lo = d16 < HALF
        key = jnp.where(m_lo, jnp.int32(0), jnp.int32(1))
        hi_dloc = d16 - HALF
        d_lo = jnp.where(m_lo, d16, HALF)
        d_hi = jnp.where(
            m_lo, HALF,
            jnp.where((hi_dloc >= 0) & (hi_dloc < HALF), hi_dloc, HALF))
        m_hi = (d16 >= HALF) & (d16 < 2 * HALF)
        s_sc[...] = s16
        dl_sc[...] = d_lo
        dh_sc[...] = d_hi
        cvec[...] = jnp.where(m_lo, jnp.int32(1), jnp.int32(0))
        m2_sc[...] = jnp.where(m_hi, jnp.int32(1), jnp.int32(0))
        vs = s_sc[...]
        vdl = dl_sc[...]
        vdh = dh_sc[...]
        vm = cvec[...]
        vm2 = m2_sc[...]
        for k in range(16):
            sspl = jnp.full((16,), vs[k], jnp.int32)
            lo_s[pl.ds(lo_off, 16)] = sspl
            lo_d[pl.ds(lo_off, 16)] = jnp.full((16,), vdl[k], jnp.int32)
            hi_s[pl.ds(hi_off, 16)] = sspl
            hi_d[pl.ds(hi_off, 16)] = jnp.full((16,), vdh[k], jnp.int32)
            lo_off = lo_off + vm[k]
            hi_off = hi_off + vm2[k]
        return (lo_off, hi_off)

    lo_cnt, hi_cnt = lax.fori_loop(
        0, PCHUNKS * (CHUNK // 16), group, (jnp.int32(0), jnp.int32(0)))

    zeros16 = jnp.zeros((16,), jnp.int32)
    dummy16 = jnp.full((16,), HALF, jnp.int32)
    for k in range(CHUNK // 16):
        lo_s[pl.ds(lo_cnt + k * 16, 16)] = zeros16
        lo_d[pl.ds(lo_cnt + k * 16, 16)] = dummy16
        hi_s[pl.ds(hi_cnt + k * 16, 16)] = zeros16
        hi_d[pl.ds(hi_cnt + k * 16, 16)] = dummy16

    nch_lo = lax.shift_right_logical(lo_cnt + (CHUNK - 1), 7)
    nch_hi = lax.shift_right_logical(hi_cnt + (CHUNK - 1), 7)
    cvec[...] = jnp.where(iota == 0, nch_lo, jnp.where(iota == 1, nch_hi, 0))

    pltpu.sync_copy(lo_s.at[pl.ds(0, LIST_LEN)], esrc_out.at[0, p])
    pltpu.sync_copy(lo_d.at[pl.ds(0, LIST_LEN)], edst_out.at[0, p])
    pltpu.sync_copy(hi_s.at[pl.ds(0, LIST_LEN)], esrc_out.at[1, p])
    pltpu.sync_copy(hi_d.at[pl.ds(0, LIST_LEN)], edst_out.at[1, p])
    pltpu.sync_copy(cvec, nch_out.at[p])


@functools.lru_cache(maxsize=None)
def _make_part():
    mesh = plsc.VectorSubcoreMesh(core_axis_name="c", subcore_axis_name="s")
    return pl.kernel(
        _part_kernel_body,
        mesh=mesh,
        out_type=(jax.ShapeDtypeStruct((NC, NW, LIST_LEN), jnp.int32),
                  jax.ShapeDtypeStruct((NC, NW, LIST_LEN), jnp.int32),
                  jax.ShapeDtypeStruct((NW, 16), jnp.int32)),
        scratch_types=[
            pltpu.VMEM((PCHUNKS, CHUNK), jnp.int32),
            pltpu.VMEM((PCHUNKS, CHUNK), jnp.int32),
            pltpu.VMEM((LIST_LEN + 16,), jnp.int32),
            pltpu.VMEM((LIST_LEN + 16,), jnp.int32),
            pltpu.VMEM((LIST_LEN + 16,), jnp.int32),
            pltpu.VMEM((LIST_LEN + 16,), jnp.int32),
            pltpu.VMEM((16,), jnp.int32),
            pltpu.VMEM((16,), jnp.int32),
            pltpu.VMEM((16,), jnp.int32),
            pltpu.VMEM((16,), jnp.int32),
            pltpu.VMEM((16,), jnp.int32),
        ],
    )


def _read_nch(nch_v, sel):
    v = nch_v[...]
    return jnp.where(sel == 0, v[0], v[1])


def _acc_zero(acc_sh, zbuf, off, n):
    done = 0
    while n - done >= CHUNK:
        pltpu.sync_copy(zbuf, acc_sh.at[pl.ds(off + done, CHUNK)])
        done += CHUNK
    if n > done:
        pltpu.sync_copy(zbuf.at[pl.ds(0, n - done)],
                        acc_sh.at[pl.ds(off + done, n - done)])


def _acc_writeback(acc_sh, buf, out_hbm, acc_off, out_off, n):
    done = 0
    while done < n:
        m = min(CHUNK, n - done)
        pltpu.sync_copy(acc_sh.at[pl.ds(acc_off + done, m)], buf.at[pl.ds(0, m)])
        pltpu.sync_copy(buf.at[pl.ds(0, m)], out_hbm.at[pl.ds(out_off + done, m)])
        done += m


def _agg_kernel_body(h_hbm, esrc, edst, nchr, out_hbm,
                     src_v, dst_v, rows_a, rows_b, rows_c, nch_v, acc_sh,
                     ga, gb, gc):
    c = lax.axis_index("c")
    s = lax.axis_index("s")
    base = s * SUB_ROWS
    node_base = c * HALF

    zbuf = rows_b

    def zb(i, _):
        for j in range(D // 16):
            zbuf[i, pl.ds(j * 16, 16)] = jnp.zeros((16,), jnp.float32)
        return 0
    lax.fori_loop(0, CHUNK, zb, 0)

    @pl.when(s < NS - 1)
    def _():
        _acc_zero(acc_sh, zbuf, base, SUB_ROWS)

    @pl.when(s == NS - 1)
    def _():
        _acc_zero(acc_sh, zbuf, base, SUB_ROWS_LAST)
        pltpu.sync_copy(zbuf.at[pl.ds(0, ACC_ROWS - HALF)],
                        acc_sh.at[pl.ds(HALF, ACC_ROWS - HALF)])

    plsc.subcore_barrier()

    for k in range(2):
        p = 2 * s + k
        pltpu.sync_copy(nchr.at[p], nch_v)
        nch = _read_nch(nch_v, c)
        pltpu.sync_copy(esrc.at[c, p], src_v)
        pltpu.sync_copy(edst.at[c, p], dst_v)

        @pl.when(nch > 0)
        def _():
            pltpu.async_copy(h_hbm.at[src_v.at[0]], rows_a, ga)

        @pl.when(nch > 1)
        def _():
            pltpu.async_copy(h_hbm.at[src_v.at[1]], rows_b, gb)

        def tri_step(jj, _):
            j0 = 3 * jj
            bufs = ((rows_a, ga), (rows_b, gb), (rows_c, gc))
            for t in range(3):
                j = j0 + t
                buf, sem = bufs[t]
                nbuf, nsem = bufs[(t + 2) % 3]

                @pl.when(j < nch)
                def _():
                    pltpu.make_async_copy(h_hbm.at[src_v.at[j]], buf, sem).wait()

                @pl.when(j + 2 < nch)
                def _():
                    pltpu.async_copy(h_hbm.at[src_v.at[j + 2]], nbuf, nsem)

                @pl.when(j < nch)
                def _():
                    pltpu.sync_copy(buf, acc_sh.at[dst_v.at[j]], add=True)
            return 0

        lax.fori_loop(0, (nch + 2) // 3, tri_step, 0)

    plsc.subcore_barrier()

    out_base = node_base + base

    @pl.when(s < NS - 1)
    def _():
        _acc_writeback(acc_sh, rows_a, out_hbm, base, out_base, SUB_ROWS)

    @pl.when(s == NS - 1)
    def _():
        _acc_writeback(acc_sh, rows_a, out_hbm, base, out_base, SUB_ROWS_LAST)


@functools.lru_cache(maxsize=None)
def _make_agg():
    mesh = plsc.VectorSubcoreMesh(core_axis_name="c", subcore_axis_name="s")
    scratch = [
        pltpu.VMEM((LIST_CH, CHUNK), jnp.int32),
        pltpu.VMEM((LIST_CH, CHUNK), jnp.int32),
        pltpu.VMEM((CHUNK, D), jnp.float32),
        pltpu.VMEM((CHUNK, D), jnp.float32),
        pltpu.VMEM((CHUNK, D), jnp.float32),
        pltpu.VMEM((16,), jnp.int32),
        pltpu.VMEM_SHARED((ACC_ROWS, D), jnp.float32),
        pltpu.SemaphoreType.DMA,
        pltpu.SemaphoreType.DMA,
        pltpu.SemaphoreType.DMA,
    ]
    return pl.kernel(
        _agg_kernel_body,
        mesh=mesh,
        out_type=jax.ShapeDtypeStruct((N_PAD, D), jnp.float32),
        scratch_types=scratch,
    )


def _cnt_kernel_body(edst, nchr, cnt0_out, cnt1_out,
                     dst_v, ones_v, zcnt, nch_v, cnt_sh, sem):
    c = lax.axis_index("c")
    s = lax.axis_index("s")

    for j in range(CHUNK // 16):
        ones_v[pl.ds(j * 16, 16)] = jnp.ones((16,), jnp.float32)

    def zc(i, _):
        zcnt[pl.ds(i * 16, 16)] = jnp.zeros((16,), jnp.float32)
        return 0
    lax.fori_loop(0, (HALF + 80) // 16, zc, 0)

    @pl.when(s == 0)
    def _():
        pltpu.sync_copy(zcnt.at[pl.ds(0, ACC_ROWS)], cnt_sh)

    plsc.subcore_barrier()

    fire_k = 8
    for k in range(2):
        p = 2 * s + k
        pltpu.sync_copy(nchr.at[p], nch_v)
        nch = _read_nch(nch_v, c)
        pltpu.sync_copy(edst.at[c, p], dst_v)

        def group(g, _):
            for kk in range(fire_k):
                @pl.when(g * fire_k + kk < nch)
                def _():
                    pltpu.async_copy(
                        ones_v, cnt_sh.at[dst_v.at[g * fire_k + kk]],
                        sem, add=True)
            for kk in range(fire_k):
                @pl.when(g * fire_k + kk < nch)
                def _():
                    pltpu.make_async_copy(
                        ones_v, cnt_sh.at[dst_v.at[g * fire_k + kk]],
                        sem).wait()
            return 0
        lax.fori_loop(0, (nch + fire_k - 1) // fire_k, group, 0)

    plsc.subcore_barrier()

    @pl.when(s == 0)
    def _():
        pltpu.sync_copy(cnt_sh.at[pl.ds(0, HALF)], zcnt.at[pl.ds(0, HALF)])

        @pl.when(c == 0)
        def _():
            pltpu.sync_copy(zcnt, cnt0_out)

        @pl.when(c == 1)
        def _():
            pltpu.sync_copy(zcnt, cnt1_out)


@functools.lru_cache(maxsize=None)
def _make_cnt():
    mesh = plsc.VectorSubcoreMesh(core_axis_name="c", subcore_axis_name="s")
    scratch = [
        pltpu.VMEM((LIST_CH, CHUNK), jnp.int32),
        pltpu.VMEM((CHUNK,), jnp.float32),
        pltpu.VMEM((HALF + 80,), jnp.float32),
        pltpu.VMEM((16,), jnp.int32),
        pltpu.VMEM_SHARED((ACC_ROWS,), jnp.float32),
        pltpu.SemaphoreType.DMA,
    ]
    return pl.kernel(
        _cnt_kernel_body,
        mesh=mesh,
        out_type=(jax.ShapeDtypeStruct((HALF + 80,), jnp.float32),
                  jax.ShapeDtypeStruct((HALF + 80,), jnp.float32)),
        scratch_types=scratch,
    )


def _tc_layer_body(relu, a_ref, cnt_ref, x_ref, wl_ref, wr_ref, b_ref, out_ref):
    rec = 1.0 / jnp.maximum(cnt_ref[...], 1.0)
    mean = a_ref[...] * rec
    z = (jnp.dot(mean, wl_ref[...], preferred_element_type=jnp.float32)
         + jnp.dot(x_ref[...], wr_ref[...], preferred_element_type=jnp.float32)
         + b_ref[...])
    out_ref[...] = jnp.maximum(z, 0.0) if relu else z


@functools.lru_cache(maxsize=None)
def _make_tc_layer(relu):
    B = 1024
    grid = (N_PAD // B,)
    return pl.pallas_call(
        functools.partial(_tc_layer_body, relu),
        grid=grid,
        in_specs=[
            pl.BlockSpec((B, D), lambda i: (i, 0)),
            pl.BlockSpec((B, 1), lambda i: (i, 0)),
            pl.BlockSpec((B, D), lambda i: (i, 0)),
            pl.BlockSpec((D, D), lambda i: (0, 0)),
            pl.BlockSpec((D, D), lambda i: (0, 0)),
            pl.BlockSpec((1, D), lambda i: (0, 0)),
        ],
        out_specs=pl.BlockSpec((B, D), lambda i: (i, 0)),
        out_shape=jax.ShapeDtypeStruct((N_PAD, D), jnp.float32),
    )


def kernel(x, edge_index, W_l1, W_r1, b1, W_l2, W_r2, b2, W_l3, W_r3, b3):
    src = edge_index[0]
    dst = edge_index[1]
    pad_e = E_PAD - E_EDGES
    src_r = jnp.concatenate(
        [src, jnp.zeros((pad_e,), jnp.int32)]).reshape(NW, PCHUNKS, CHUNK)
    dst_r = jnp.concatenate(
        [dst, jnp.full((pad_e,), PAD_DST, jnp.int32)]).reshape(NW, PCHUNKS, CHUNK)

    x_pad = jnp.pad(x, ((0, N_PAD - N_NODES), (0, 0)))

    esrc1, edst1, nch = _make_part()(src_r, dst_r)
    esrc = esrc1.reshape(NC, NW, LIST_CH, CHUNK)
    edst = edst1.reshape(NC, NW, LIST_CH, CHUNK)

    cnt0, cnt1 = _make_cnt()(edst, nch)
    cnt = jnp.concatenate(
        [cnt0[:HALF], cnt1[:HALF],
         jnp.zeros((N_PAD - 2 * HALF,), jnp.float32)]).reshape(N_PAD, 1)

    def pad_w(w):
        return jnp.pad(w, ((0, 0), (0, D - w.shape[1])))

    b1r = b1.reshape(1, D)
    b2r = b2.reshape(1, D)
    b3r = jnp.pad(b3, (0, D - b3.shape[0])).reshape(1, D)

    agg = _make_agg()
    tc_relu = _make_tc_layer(True)

    a1 = agg(x_pad, esrc, edst, nch)
    h1 = tc_relu(a1, cnt, x_pad, W_l1, W_r1, b1r)
    a2 = agg(h1, esrc, edst, nch)
    h2 = tc_relu(a2, cnt, h1, W_l2, W_r2, b2r)
    a3 = agg(h2, esrc, edst, nch)
    out = _make_tc_layer(False)(a3, cnt, h2, pad_w(W_l3), pad_w(W_r3), b3r)
    return out[:N_NODES, :47]


# speedup vs baseline: 3.1572x; 1.0081x over previous
import functools

import jax
import jax.numpy as jnp
from jax import lax
from jax.experimental import pallas as pl
from jax.experimental.pallas import tpu as pltpu
from jax.experimental.pallas import tpu_sc as plsc

NC = 2
NS = 16
NW = NC * NS

N_NODES = 10000
HALF = 5040
ACC_ROWS = HALF + 8
SUB_ROWS = 320
SUB_ROWS_LAST = HALF - (NS - 1) * SUB_ROWS
N_PAD = 10240
E_EDGES = 320000
CHUNK = 128
PCHUNKS = 80
E_PAD = NW * PCHUNKS * CHUNK
LIST_CH = PCHUNKS + 1
LIST_LEN = LIST_CH * CHUNK
PAD_DST = 1 << 20
D = 128


def _prefix_sum(scratch, x, iota):
    for sh in (1, 2, 4, 8):
        scratch[...] = x
        sv = plsc.load_gather(scratch, [jnp.maximum(iota - sh, 0)])
        x = x + jnp.where(iota >= sh, sv, 0)
    return x


def _splat_last(scratch, x):
    scratch[...] = x
    return plsc.load_gather(scratch, [jnp.full((16,), 15, jnp.int32)])


def _part_kernel_body(srcr, dstr, esrc_out, edst_out, nch_out,
                      src_v, dst_v, lo_s, lo_d, hi_s, hi_d, cvec,
                      s_sc, dl_sc, dh_sc, m2_sc):
    c = lax.axis_index("c")
    s = lax.axis_index("s")
    p = c * NS + s

    pltpu.sync_copy(srcr.at[p], src_v)
    pltpu.sync_copy(dstr.at[p], dst_v)

    iota = lax.iota(jnp.int32, 16)

    def group(g, offs):
        lo_off, hi_off = offs
        i = g // (CHUNK // 16)
        jj = lax.rem(g, CHUNK // 16)
        sl = pl.ds(jj * 16, 16)
        s16 = src_v[i, sl]
        d16 = dst_v[i, sl]
        m_lo = d16 < HALF
        key = jnp.where(m_lo, jnp.int32(0), jnp.int32(1))
        hi_dloc = d16 - HALF
        d_lo = jnp.where(m_lo, d16, HALF)
        d_hi = jnp.where(
            m_lo, HALF,
            jnp.where((hi_dloc >= 0) & (hi_dloc < HALF), hi_dloc, HALF))
        m_hi = (d16 >= HALF) & (d16 < 2 * HALF)
        s_sc[...] = s16
        dl_sc[...] = d_lo
        dh_sc[...] = d_hi
        cvec[...] = jnp.where(m_lo, jnp.int32(1), jnp.int32(0))
        m2_sc[...] = jnp.where(m_hi, jnp.int32(1), jnp.int32(0))
        vs = s_sc[...]
        vdl = dl_sc[...]
        vdh = dh_sc[...]
        vm = cvec[...]
        vm2 = m2_sc[...]
        for k in range(16):
            sspl = jnp.full((16,), vs[k], jnp.int32)
            lo_s[pl.ds(lo_off, 16)] = sspl
            lo_d[pl.ds(lo_off, 16)] = jnp.full((16,), vdl[k], jnp.int32)
            hi_s[pl.ds(hi_off, 16)] = sspl
            hi_d[pl.ds(hi_off, 16)] = jnp.full((16,), vdh[k], jnp.int32)
            lo_off = lo_off + vm[k]
            hi_off = hi_off + vm2[k]
        return (lo_off, hi_off)

    lo_cnt, hi_cnt = lax.fori_loop(
        0, PCHUNKS * (CHUNK // 16), group, (jnp.int32(0), jnp.int32(0)))

    zeros16 = jnp.zeros((16,), jnp.int32)
    dummy16 = jnp.full((16,), HALF, jnp.int32)
    for k in range(CHUNK // 16):
        lo_s[pl.ds(lo_cnt + k * 16, 16)] = zeros16
        lo_d[pl.ds(lo_cnt + k * 16, 16)] = dummy16
        hi_s[pl.ds(hi_cnt + k * 16, 16)] = zeros16
        hi_d[pl.ds(hi_cnt + k * 16, 16)] = dummy16

    nch_lo = lax.shift_right_logical(lo_cnt + (CHUNK - 1), 7)
    nch_hi = lax.shift_right_logical(hi_cnt + (CHUNK - 1), 7)
    cvec[...] = jnp.where(iota == 0, nch_lo, jnp.where(iota == 1, nch_hi, 0))

    pltpu.sync_copy(lo_s.at[pl.ds(0, LIST_LEN)], esrc_out.at[0, p])
    pltpu.sync_copy(lo_d.at[pl.ds(0, LIST_LEN)], edst_out.at[0, p])
    pltpu.sync_copy(hi_s.at[pl.ds(0, LIST_LEN)], esrc_out.at[1, p])
    pltpu.sync_copy(hi_d.at[pl.ds(0, LIST_LEN)], edst_out.at[1, p])
    pltpu.sync_copy(cvec, nch_out.at[p])


@functools.lru_cache(maxsize=None)
def _make_part():
    mesh = plsc.VectorSubcoreMesh(core_axis_name="c", subcore_axis_name="s")
    return pl.kernel(
        _part_kernel_body,
        mesh=mesh,
        out_type=(jax.ShapeDtypeStruct((NC, NW, LIST_LEN), jnp.int32),
                  jax.ShapeDtypeStruct((NC, NW, LIST_LEN), jnp.int32),
                  jax.ShapeDtypeStruct((NW, 16), jnp.int32)),
        scratch_types=[
            pltpu.VMEM((PCHUNKS, CHUNK), jnp.int32),
            pltpu.VMEM((PCHUNKS, CHUNK), jnp.int32),
            pltpu.VMEM((LIST_LEN + 16,), jnp.int32),
            pltpu.VMEM((LIST_LEN + 16,), jnp.int32),
            pltpu.VMEM((LIST_LEN + 16,), jnp.int32),
            pltpu.VMEM((LIST_LEN + 16,), jnp.int32),
            pltpu.VMEM((16,), jnp.int32),
            pltpu.VMEM((16,), jnp.int32),
            pltpu.VMEM((16,), jnp.int32),
            pltpu.VMEM((16,), jnp.int32),
            pltpu.VMEM((16,), jnp.int32),
        ],
    )


def _read_nch(nch_v, sel):
    v = nch_v[...]
    return jnp.where(sel == 0, v[0], v[1])


def _acc_zero(acc_sh, zbuf, off, n):
    done = 0
    while n - done >= CHUNK:
        pltpu.sync_copy(zbuf, acc_sh.at[pl.ds(off + done, CHUNK)])
        done += CHUNK
    if n > done:
        pltpu.sync_copy(zbuf.at[pl.ds(0, n - done)],
                        acc_sh.at[pl.ds(off + done, n - done)])


def _acc_writeback(acc_sh, buf, out_hbm, acc_off, out_off, n):
    done = 0
    while done < n:
        m = min(CHUNK, n - done)
        pltpu.sync_copy(acc_sh.at[pl.ds(acc_off + done, m)], buf.at[pl.ds(0, m)])
        pltpu.sync_copy(buf.at[pl.ds(0, m)], out_hbm.at[pl.ds(out_off + done, m)])
        done += m


def _agg_kernel_body(h_hbm, esrc, edst, nchr, out_hbm,
                     src_v, dst_v, rows_a, rows_b, rows_c, rows_d, nch_v,
                     acc_sh, ga, gb, gc, gd):
    c = lax.axis_index("c")
    s = lax.axis_index("s")
    base = s * SUB_ROWS
    node_base = c * HALF

    zbuf = rows_b

    def zb(i, _):
        for j in range(D // 16):
            zbuf[i, pl.ds(j * 16, 16)] = jnp.zeros((16,), jnp.float32)
        return 0
    lax.fori_loop(0, CHUNK, zb, 0)

    @pl.when(s < NS - 1)
    def _():
        _acc_zero(acc_sh, zbuf, base, SUB_ROWS)

    @pl.when(s == NS - 1)
    def _():
        _acc_zero(acc_sh, zbuf, base, SUB_ROWS_LAST)
        pltpu.sync_copy(zbuf.at[pl.ds(0, ACC_ROWS - HALF)],
                        acc_sh.at[pl.ds(HALF, ACC_ROWS - HALF)])

    plsc.subcore_barrier()

    for k in range(2):
        p = 2 * s + k
        pltpu.sync_copy(nchr.at[p], nch_v)
        nch = _read_nch(nch_v, c)
        pltpu.sync_copy(esrc.at[c, p], src_v)
        pltpu.sync_copy(edst.at[c, p], dst_v)

        bufs = ((rows_a, ga), (rows_b, gb), (rows_c, gc), (rows_d, gd))
        nb = 4
        for t in range(nb - 1):
            @pl.when(nch > t)
            def _():
                pltpu.async_copy(h_hbm.at[src_v.at[t]], bufs[t][0], bufs[t][1])

        def ring_step(jj, _):
            j0 = nb * jj
            for t in range(nb):
                j = j0 + t
                buf, sem = bufs[t]
                nbuf, nsem = bufs[(t + nb - 1) % nb]

                @pl.when(j < nch)
                def _():
                    pltpu.make_async_copy(h_hbm.at[src_v.at[j]], buf, sem).wait()

                @pl.when(j + nb - 1 < nch)
                def _():
                    pltpu.async_copy(h_hbm.at[src_v.at[j + nb - 1]], nbuf, nsem)

                @pl.when(j < nch)
                def _():
                    pltpu.sync_copy(buf, acc_sh.at[dst_v.at[j]], add=True)
            return 0

        lax.fori_loop(0, (nch + nb - 1) // nb, ring_step, 0)

    plsc.subcore_barrier()

    out_base = node_base + base

    @pl.when(s < NS - 1)
    def _():
        _acc_writeback(acc_sh, rows_a, out_hbm, base, out_base, SUB_ROWS)

    @pl.when(s == NS - 1)
    def _():
        _acc_writeback(acc_sh, rows_a, out_hbm, base, out_base, SUB_ROWS_LAST)


@functools.lru_cache(maxsize=None)
def _make_agg():
    mesh = plsc.VectorSubcoreMesh(core_axis_name="c", subcore_axis_name="s")
    scratch = [
        pltpu.VMEM((LIST_CH, CHUNK), jnp.int32),
        pltpu.VMEM((LIST_CH, CHUNK), jnp.int32),
        pltpu.VMEM((CHUNK, D), jnp.float32),
        pltpu.VMEM((CHUNK, D), jnp.float32),
        pltpu.VMEM((CHUNK, D), jnp.float32),
        pltpu.VMEM((CHUNK, D), jnp.float32),
        pltpu.VMEM((16,), jnp.int32),
        pltpu.VMEM_SHARED((ACC_ROWS, D), jnp.float32),
        pltpu.SemaphoreType.DMA,
        pltpu.SemaphoreType.DMA,
        pltpu.SemaphoreType.DMA,
        pltpu.SemaphoreType.DMA,
    ]
    return pl.kernel(
        _agg_kernel_body,
        mesh=mesh,
        out_type=jax.ShapeDtypeStruct((N_PAD, D), jnp.float32),
        scratch_types=scratch,
    )


def _cnt_kernel_body(edst, nchr, cnt0_out, cnt1_out,
                     dst_v, ones_v, zcnt, nch_v, cnt_sh, sem):
    c = lax.axis_index("c")
    s = lax.axis_index("s")

    for j in range(CHUNK // 16):
        ones_v[pl.ds(j * 16, 16)] = jnp.ones((16,), jnp.float32)

    def zc(i, _):
        zcnt[pl.ds(i * 16, 16)] = jnp.zeros((16,), jnp.float32)
        return 0
    lax.fori_loop(0, (HALF + 80) // 16, zc, 0)

    @pl.when(s == 0)
    def _():
        pltpu.sync_copy(zcnt.at[pl.ds(0, ACC_ROWS)], cnt_sh)

    plsc.subcore_barrier()

    fire_k = 8
    for k in range(2):
        p = 2 * s + k
        pltpu.sync_copy(nchr.at[p], nch_v)
        nch = _read_nch(nch_v, c)
        pltpu.sync_copy(edst.at[c, p], dst_v)

        def group(g, _):
            for kk in range(fire_k):
                @pl.when(g * fire_k + kk < nch)
                def _():
                    pltpu.async_copy(
                        ones_v, cnt_sh.at[dst_v.at[g * fire_k + kk]],
                        sem, add=True)
            for kk in range(fire_k):
                @pl.when(g * fire_k + kk < nch)
                def _():
                    pltpu.make_async_copy(
                        ones_v, cnt_sh.at[dst_v.at[g * fire_k + kk]],
                        sem).wait()
            return 0
        lax.fori_loop(0, (nch + fire_k - 1) // fire_k, group, 0)

    plsc.subcore_barrier()

    @pl.when(s == 0)
    def _():
        pltpu.sync_copy(cnt_sh.at[pl.ds(0, HALF)], zcnt.at[pl.ds(0, HALF)])

        @pl.when(c == 0)
        def _():
            pltpu.sync_copy(zcnt, cnt0_out)

        @pl.when(c == 1)
        def _():
            pltpu.sync_copy(zcnt, cnt1_out)


@functools.lru_cache(maxsize=None)
def _make_cnt():
    mesh = plsc.VectorSubcoreMesh(core_axis_name="c", subcore_axis_name="s")
    scratch = [
        pltpu.VMEM((LIST_CH, CHUNK), jnp.int32),
        pltpu.VMEM((CHUNK,), jnp.float32),
        pltpu.VMEM((HALF + 80,), jnp.float32),
        pltpu.VMEM((16,), jnp.int32),
        pltpu.VMEM_SHARED((ACC_ROWS,), jnp.float32),
        pltpu.SemaphoreType.DMA,
    ]
    return pl.kernel(
        _cnt_kernel_body,
        mesh=mesh,
        out_type=(jax.ShapeDtypeStruct((HALF + 80,), jnp.float32),
                  jax.ShapeDtypeStruct((HALF + 80,), jnp.float32)),
        scratch_types=scratch,
    )


def _tc_layer_body(relu, a_ref, cnt_ref, x_ref, wl_ref, wr_ref, b_ref, out_ref):
    rec = 1.0 / jnp.maximum(cnt_ref[...], 1.0)
    mean = a_ref[...] * rec
    z = (jnp.dot(mean, wl_ref[...], preferred_element_type=jnp.float32)
         + jnp.dot(x_ref[...], wr_ref[...], preferred_element_type=jnp.float32)
         + b_ref[...])
    out_ref[...] = jnp.maximum(z, 0.0) if relu else z


@functools.lru_cache(maxsize=None)
def _make_tc_layer(relu):
    B = 1024
    grid = (N_PAD // B,)
    return pl.pallas_call(
        functools.partial(_tc_layer_body, relu),
        grid=grid,
        in_specs=[
            pl.BlockSpec((B, D), lambda i: (i, 0)),
            pl.BlockSpec((B, 1), lambda i: (i, 0)),
            pl.BlockSpec((B, D), lambda i: (i, 0)),
            pl.BlockSpec((D, D), lambda i: (0, 0)),
            pl.BlockSpec((D, D), lambda i: (0, 0)),
            pl.BlockSpec((1, D), lambda i: (0, 0)),
        ],
        out_specs=pl.BlockSpec((B, D), lambda i: (i, 0)),
        out_shape=jax.ShapeDtypeStruct((N_PAD, D), jnp.float32),
    )


def kernel(x, edge_index, W_l1, W_r1, b1, W_l2, W_r2, b2, W_l3, W_r3, b3):
    src = edge_index[0]
    dst = edge_index[1]
    pad_e = E_PAD - E_EDGES
    src_r = jnp.concatenate(
        [src, jnp.zeros((pad_e,), jnp.int32)]).reshape(NW, PCHUNKS, CHUNK)
    dst_r = jnp.concatenate(
        [dst, jnp.full((pad_e,), PAD_DST, jnp.int32)]).reshape(NW, PCHUNKS, CHUNK)

    x_pad = jnp.pad(x, ((0, N_PAD - N_NODES), (0, 0)))

    esrc1, edst1, nch = _make_part()(src_r, dst_r)
    esrc = esrc1.reshape(NC, NW, LIST_CH, CHUNK)
    edst = edst1.reshape(NC, NW, LIST_CH, CHUNK)

    cnt0, cnt1 = _make_cnt()(edst, nch)
    cnt = jnp.concatenate(
        [cnt0[:HALF], cnt1[:HALF],
         jnp.zeros((N_PAD - 2 * HALF,), jnp.float32)]).reshape(N_PAD, 1)

    def pad_w(w):
        return jnp.pad(w, ((0, 0), (0, D - w.shape[1])))

    b1r = b1.reshape(1, D)
    b2r = b2.reshape(1, D)
    b3r = jnp.pad(b3, (0, D - b3.shape[0])).reshape(1, D)

    agg = _make_agg()
    tc_relu = _make_tc_layer(True)

    a1 = agg(x_pad, esrc, edst, nch)
    h1 = tc_relu(a1, cnt, x_pad, W_l1, W_r1, b1r)
    a2 = agg(h1, esrc, edst, nch)
    h2 = tc_relu(a2, cnt, h1, W_l2, W_r2, b2r)
    a3 = agg(h2, esrc, edst, nch)
    out = _make_tc_layer(False)(a3, cnt, h2, pad_w(W_l3), pad_w(W_r3), b3r)
    return out[:N_NODES, :47]
